# speedup vs baseline: 1.0074x; 1.0074x over previous
import jax
import jax.numpy as jnp
from jax.experimental import pallas as pl


def _affine_kernel(h_ref, a_ref, b_ref, o_ref):
    o_ref[...] = h_ref[...] * a_ref[...] + b_ref[...]


def _affine(h, a, b):
    return pl.pallas_call(
        _affine_kernel,
        out_shape=jax.ShapeDtypeStruct(h.shape, h.dtype),
    )(h, a[None, :], b[None, :])


def _gn_consts(s1, s2, p, n):
    mean = s1 / n
    var = s2 / n - 2.0 * mean * p["ms"] * mean + (mean * p["ms"]) ** 2
    alpha = p["w"] / jnp.sqrt(var + 1e-6)
    beta = p["b"] - alpha * mean * p["ms"]
    return alpha, beta


def kernel(x, edge_index, edge_weight, emb_table, emb_gn, conv_gns, layer_gns):
    n_node = x.shape[0]
    row, col = edge_index[0], edge_index[1]
    deg = jax.ops.segment_sum(edge_weight, row, num_segments=n_node)
    deg = jnp.where(deg < 0.5, deg + 1.0, deg)
    w = (1.0 / deg)[row] * edge_weight

    def spmm(h):
        return jax.ops.segment_sum(w[:, None] * h[col], row, num_segments=n_node)

    def norm2(h, p1, p2):
        s1 = jnp.sum(h, axis=0)
        s2 = jnp.sum(h * h, axis=0)
        a1, b1 = _gn_consts(s1, s2, p1, n_node)
        s1b = a1 * s1 + n_node * b1
        s2b = a1 * a1 * s2 + 2.0 * a1 * b1 * s1 + n_node * b1 * b1
        a2, b2 = _gn_consts(s1b, s2b, p2, n_node)
        return _affine(h, a2 * a1, a2 * b1 + b2)

    h = jnp.take(emb_table, x, axis=0)
    s1 = jnp.sum(h, axis=0)
    s2 = jnp.sum(h * h, axis=0)
    a0, b0 = _gn_consts(s1, s2, emb_gn, n_node)
    h = _affine(h, a0, b0)
    for layer in range(2):
        h = spmm(h)
        h = norm2(h, conv_gns[layer], layer_gns[layer])
        h = jax.nn.relu(h)
    h = spmm(h)
    h = norm2(h, conv_gns[2], layer_gns[2])
    return h


# speedup vs baseline: 5.2204x; 5.1823x over previous
import functools

import jax
import jax.numpy as jnp
from jax import lax
from jax.experimental import pallas as pl
from jax.experimental.pallas import tpu as pltpu
from jax.experimental.pallas import tpu_sc as plsc

N = 10000
NPAD = 10240
E = 320000
HH = 64
NT = 16
RPT = NPAD // NT
RVAL = N - 15 * RPT
EPT = E // NT
WD = 2000
WE = 400
NWD = EPT // WD
NWE = EPT // WE
EPS = 1e-6


def _mesh():
    return plsc.VectorSubcoreMesh(
        core_axis_name="c", subcore_axis_name="s", num_cores=2, num_subcores=16
    )


def _rsqrt16(v):
    i = lax.bitcast_convert_type(v, jnp.int32)
    i = jnp.int32(0x5F3759DF) - lax.shift_right_logical(i, 1)
    y = lax.bitcast_convert_type(i, jnp.float32)
    for _ in range(4):
        y = y * (1.5 - 0.5 * v * y * y)
    return y


def _deg_body(row_hbm, ew_hbm, inv_hbm, idx_v, val_v, deg_sp):
    cid = lax.axis_index("c")
    tid = lax.axis_index("s")

    def zero(i, c):
        val_v[pl.ds(i * 16, 16)] = jnp.zeros((16,), jnp.float32)
        return c
    lax.fori_loop(0, RPT // 16, zero, 0)
    pltpu.sync_copy(val_v.at[pl.ds(0, RPT)], deg_sp.at[pl.ds(tid * RPT, RPT)])
    plsc.subcore_barrier()

    def win(wi, c):
        base = tid * EPT + wi * WD
        pltpu.sync_copy(row_hbm.at[pl.ds(base, WD)], idx_v)
        pltpu.sync_copy(ew_hbm.at[pl.ds(base, WD)], val_v)
        pltpu.sync_copy(val_v, deg_sp.at[idx_v], add=True)
        return c
    lax.fori_loop(0, NWD, win, 0)
    plsc.subcore_barrier()

    pltpu.sync_copy(deg_sp.at[pl.ds(tid * RPT, RPT)], val_v.at[pl.ds(0, RPT)])

    def inv(i, c):
        d = val_v[pl.ds(i * 16, 16)]
        d = jnp.where(d < 0.5, d + 1.0, d)
        val_v[pl.ds(i * 16, 16)] = 1.0 / d
        return c
    lax.fori_loop(0, RPT // 16, inv, 0)

    @pl.when(cid == 0)
    def _():
        pltpu.sync_copy(val_v.at[pl.ds(0, RPT)], inv_hbm.at[pl.ds(tid * RPT, RPT)])


def _deg_pass(row, ew):
    return pl.kernel(
        _deg_body,
        out_type=jax.ShapeDtypeStruct((NPAD,), jnp.float32),
        mesh=_mesh(),
        compiler_params=pltpu.CompilerParams(needs_layout_passes=False, use_tc_tiling_on_sc=False),
        scratch_types=[
            pltpu.VMEM((WD,), jnp.int32),
            pltpu.VMEM((WD,), jnp.float32),
            pltpu.VMEM_SHARED((NPAD,), jnp.float32),
        ],
    )(row, ew)


def _stats_reduce_and_consts(stats_sp, part_v, stat_v, s1s, s2s, tid):
    for c in range(4):
        part_v[0, pl.ds(c * 16, 16)] = s1s[c]
        part_v[1, pl.ds(c * 16, 16)] = s2s[c]
    pltpu.sync_copy(part_v.at[pl.ds(0, 1)], stats_sp.at[pl.ds(tid, 1)])
    pltpu.sync_copy(part_v.at[pl.ds(1, 1)], stats_sp.at[pl.ds(NT + tid, 1)])
    plsc.subcore_barrier()

    @pl.when(tid == 0)
    def _():
        pltpu.sync_copy(stats_sp.at[pl.ds(0, 2 * NT)], part_v.at[pl.ds(2, 2 * NT)])
        for c in range(4):
            def red(i, acc):
                a1, a2 = acc
                return (a1 + part_v[2 + i, pl.ds(c * 16, 16)],
                        a2 + part_v[2 + NT + i, pl.ds(c * 16, 16)])
            z = jnp.zeros((16,), jnp.float32)
            t1, t2 = lax.fori_loop(0, NT, red, (z, z))
            part_v[0, pl.ds(c * 16, 16)] = t1
            part_v[1, pl.ds(c * 16, 16)] = t2
        pltpu.sync_copy(part_v.at[pl.ds(0, 2)], stats_sp.at[pl.ds(2 * NT, 2)])
    plsc.subcore_barrier()
    pltpu.sync_copy(stats_sp.at[pl.ds(2 * NT, 2)], stat_v)


def _emb_body(x_hbm, tbl_hbm, gnp_hbm, h_hbm,
              tbl_v, xs_v, rows_v, part_v, stat_v, gn_v, stats_sp, sem):
    cid = lax.axis_index("c")
    tid = lax.axis_index("s")
    pltpu.sync_copy(tbl_hbm, tbl_v)
    pltpu.sync_copy(x_hbm.at[pl.ds(tid * RPT, RPT)], xs_v)
    roff = cid * 65

    def node(k, carry):
        s1s, s2s = carry
        f = jnp.where(tid * RPT + k < N, 1.0, 0.0)
        rowi = plsc.load_gather(xs_v, [jnp.full((16,), k, jnp.int32)]) + roff
        n1, n2 = [], []
        for c in range(4):
            cols = lax.iota(jnp.int32, 16) + c * 16
            r = plsc.load_gather(tbl_v, [rowi, cols])
            rows_v[k, pl.ds(c * 16, 16)] = r
            rf = r * f
            n1.append(s1s[c] + rf)
            n2.append(s2s[c] + rf * r)
        return tuple(n1), tuple(n2)

    z = jnp.zeros((16,), jnp.float32)
    s1s, s2s = lax.fori_loop(0, RPT, node, ((z,) * 4, (z,) * 4))

    _stats_reduce_and_consts(stats_sp, part_v, stat_v, s1s, s2s, tid)
    pltpu.sync_copy(gnp_hbm.at[pl.ds(cid * 8, 8)], gn_v)

    alphas, betas = [], []
    for c in range(4):
        cs = pl.ds(c * 16, 16)
        s1, s2 = stat_v[0, cs], stat_v[1, cs]
        w, b, ms = gn_v[0, cs], gn_v[1, cs], gn_v[2, cs]
        m = s1 * (1.0 / N)
        var = s2 * (1.0 / N) - m * m * ms * (2.0 - ms)
        a = w * _rsqrt16(var + EPS)
        alphas.append(a)
        betas.append(b - a * m * ms)

    def app(k, c):
        for ci in range(4):
            cs = pl.ds(ci * 16, 16)
            rows_v[k, cs] = rows_v[k, cs] * alphas[ci] + betas[ci]
        return c
    lax.fori_loop(0, RPT, app, 0)

    base = cid * N + tid * RPT

    @pl.when(tid < NT - 1)
    def _():
        pltpu.sync_copy(rows_v, h_hbm.at[pl.ds(base, RPT)])

    @pl.when(tid == NT - 1)
    def _():
        pltpu.sync_copy(rows_v.at[pl.ds(0, RVAL)], h_hbm.at[pl.ds(base, RVAL)])


def _emb_pass(x_pad, tbl2, gnp):
    return pl.kernel(
        _emb_body,
        out_type=jax.ShapeDtypeStruct((2 * N, HH), jnp.float32),
        mesh=_mesh(),
        compiler_params=pltpu.CompilerParams(needs_layout_passes=False, use_tc_tiling_on_sc=False),
        scratch_types=[
            pltpu.VMEM((130, HH), jnp.float32),
            pltpu.VMEM((RPT,), jnp.int32),
            pltpu.VMEM((RPT, HH), jnp.float32),
            pltpu.VMEM((2 + 2 * NT, HH), jnp.float32),
            pltpu.VMEM((2, HH), jnp.float32),
            pltpu.VMEM((8, HH), jnp.float32),
            pltpu.VMEM_SHARED((2 * NT + 2, HH), jnp.float32),
            pltpu.SemaphoreType.DMA,
        ],
    )(x_pad, tbl2, gnp)


def _spmm_body(h_in, row_hbm, col_hbm, ew_hbm, inv_hbm, gnp_hbm, h_out,
               col_v, row_v, ew_v, grows_v, rows_v, inv_v,
               part_v, stat_v, gn_v, acc_sp, stats_sp, sem):
    cid = lax.axis_index("c")
    tid = lax.axis_index("s")

    def zrow(k, c):
        for ci in range(4):
            rows_v[k, pl.ds(ci * 16, 16)] = jnp.zeros((16,), jnp.float32)
        return c
    lax.fori_loop(0, RPT, zrow, 0)
    pltpu.sync_copy(rows_v, acc_sp.at[pl.ds(tid * RPT, RPT)])
    plsc.subcore_barrier()

    coff = cid * N

    def win(wi, c):
        base = tid * EPT + wi * WE
        pltpu.sync_copy(col_hbm.at[pl.ds(base, WE)], col_v)
        pltpu.sync_copy(row_hbm.at[pl.ds(base, WE)], row_v)
        pltpu.sync_copy(ew_hbm.at[pl.ds(base, WE)], ew_v)

        def adj(i, c2):
            col_v[pl.ds(i * 16, 16)] = col_v[pl.ds(i * 16, 16)] + coff
            return c2
        lax.fori_loop(0, WE // 16, adj, 0)
        pltpu.async_copy(h_in.at[col_v], grows_v, sem).wait()

        def mul(k, c2):
            wk = plsc.load_gather(ew_v, [jnp.full((16,), k, jnp.int32)])
            for ci in range(4):
                cs = pl.ds(ci * 16, 16)
                grows_v[k, cs] = grows_v[k, cs] * wk
            return c2
        lax.fori_loop(0, WE, mul, 0)
        pltpu.sync_copy(grows_v, acc_sp.at[row_v], add=True)
        return c
    lax.fori_loop(0, NWE, win, 0)
    plsc.subcore_barrier()

    r0 = tid * RPT
    pltpu.sync_copy(acc_sp.at[pl.ds(r0, RPT)], rows_v)
    pltpu.sync_copy(inv_hbm.at[pl.ds(r0, RPT)], inv_v)

    def srow(k, carry):
        s1s, s2s = carry
        f = jnp.where(r0 + k < N, 1.0, 0.0)
        s = plsc.load_gather(inv_v, [jnp.full((16,), k, jnp.int32)]) * f
        n1, n2 = [], []
        for ci in range(4):
            cs = pl.ds(ci * 16, 16)
            t = rows_v[k, cs] * s
            rows_v[k, cs] = t
            n1.append(s1s[ci] + t)
            n2.append(s2s[ci] + t * t)
        return tuple(n1), tuple(n2)

    z = jnp.zeros((16,), jnp.float32)
    s1s, s2s = lax.fori_loop(0, RPT, srow, ((z,) * 4, (z,) * 4))

    _stats_reduce_and_consts(stats_sp, part_v, stat_v, s1s, s2s, tid)
    pltpu.sync_copy(gnp_hbm.at[pl.ds(cid * 8, 8)], gn_v)

    alphas, betas, caps = [], [], []
    for c in range(4):
        cs = pl.ds(c * 16, 16)
        s1, s2 = stat_v[0, cs], stat_v[1, cs]
        w1, b1, ms1 = gn_v[0, cs], gn_v[1, cs], gn_v[2, cs]
        w2, b2, ms2 = gn_v[3, cs], gn_v[4, cs], gn_v[5, cs]
        cap = gn_v[6, cs]
        m = s1 * (1.0 / N)
        var1 = s2 * (1.0 / N) - m * m * ms1 * (2.0 - ms1)
        a1 = w1 * _rsqrt16(var1 + EPS)
        b1f = b1 - a1 * m * ms1
        m2 = a1 * m + b1f
        varh = s2 * (1.0 / N) - m * m
        mo = m2 * (1.0 - ms2)
        var2 = a1 * a1 * varh + mo * mo
        a2 = w2 * _rsqrt16(var2 + EPS)
        b2f = b2 - a2 * m2 * ms2
        alphas.append(a2 * a1)
        betas.append(a2 * b1f + b2f)
        caps.append(cap)

    def app(k, c):
        for ci in range(4):
            cs = pl.ds(ci * 16, 16)
            t = rows_v[k, cs] * alphas[ci] + betas[ci]
            rows_v[k, cs] = jnp.maximum(t, caps[ci])
        return c
    lax.fori_loop(0, RPT, app, 0)

    base = cid * N + r0

    @pl.when(tid < NT - 1)
    def _():
        pltpu.sync_copy(rows_v, h_out.at[pl.ds(base, RPT)])

    @pl.when(tid == NT - 1)
    def _():
        pltpu.sync_copy(rows_v.at[pl.ds(0, RVAL)], h_out.at[pl.ds(base, RVAL)])


def _spmm_pass(h, row, col, ew, inv, gnp):
    return pl.kernel(
        _spmm_body,
        out_type=jax.ShapeDtypeStruct((2 * N, HH), jnp.float32),
        mesh=_mesh(),
        compiler_params=pltpu.CompilerParams(needs_layout_passes=False, use_tc_tiling_on_sc=False),
        scratch_types=[
            pltpu.VMEM((WE,), jnp.int32),
            pltpu.VMEM((WE,), jnp.int32),
            pltpu.VMEM((WE,), jnp.float32),
            pltpu.VMEM((WE, HH), jnp.float32),
            pltpu.VMEM((RPT, HH), jnp.float32),
            pltpu.VMEM((RPT,), jnp.float32),
            pltpu.VMEM((2 + 2 * NT, HH), jnp.float32),
            pltpu.VMEM((2, HH), jnp.float32),
            pltpu.VMEM((8, HH), jnp.float32),
            pltpu.VMEM_SHARED((NPAD, HH), jnp.float32),
            pltpu.VMEM_SHARED((2 * NT + 2, HH), jnp.float32),
            pltpu.SemaphoreType.DMA,
        ],
    )(h, row, col, ew, inv, gnp)


def kernel(x, edge_index, edge_weight, emb_table, emb_gn, conv_gns, layer_gns):
    row = edge_index[0]
    col = edge_index[1]
    x_pad = jnp.concatenate([x, jnp.zeros((NPAD - N,), jnp.int32)])
    tbl2 = jnp.concatenate([emb_table[:, :HH], emb_table[:, HH:]], axis=0)
    zrow = jnp.zeros((HH,), jnp.float32)
    gnp_emb = jnp.stack(
        [emb_gn["w"][:HH], emb_gn["b"][:HH], emb_gn["ms"][:HH]]
        + [zrow] * 5
        + [emb_gn["w"][HH:], emb_gn["b"][HH:], emb_gn["ms"][HH:]]
        + [zrow] * 5)

    inv = _deg_pass(row, edge_weight)
    h = _emb_pass(x_pad, tbl2, gnp_emb)
    gnps = []
    for layer in range(3):
        p1, p2 = conv_gns[layer], layer_gns[layer]
        cap = jnp.full((HH,), 0.0 if layer < 2 else -3.4e38, jnp.float32)
        rows = []
        for half in (slice(0, HH), slice(HH, 2 * HH)):
            for p in (p1, p2):
                rows += [p["w"][half], p["b"][half], p["ms"][half]]
            rows += [cap, zrow]
        gnps.append(jnp.stack(rows))

    def step(h, gnp):
        return _spmm_pass(h, row, col, edge_weight, inv, gnp), None

    h, _ = lax.scan(step, h, jnp.stack(gnps))
    return jnp.concatenate([h[:N], h[N:]], axis=1)


# speedup vs baseline: 7.8231x; 1.4986x over previous
import functools

import jax
import jax.numpy as jnp
from jax import lax
from jax.experimental import pallas as pl
from jax.experimental.pallas import tpu as pltpu
from jax.experimental.pallas import tpu_sc as plsc

N = 10000
NPAD = 10240
E = 320000
HH = 64
NT = 16
RPT = NPAD // NT
RVAL = N - 15 * RPT
EPT = E // NT
WD = 2000
WE = 400
NWD = EPT // WD
NWE = EPT // WE
EPS = 1e-6


def _mesh():
    return plsc.VectorSubcoreMesh(
        core_axis_name="c", subcore_axis_name="s", num_cores=2, num_subcores=16
    )


def _rsqrt16(v):
    i = lax.bitcast_convert_type(v, jnp.int32)
    i = jnp.int32(0x5F3759DF) - lax.shift_right_logical(i, 1)
    y = lax.bitcast_convert_type(i, jnp.float32)
    for _ in range(4):
        y = y * (1.5 - 0.5 * v * y * y)
    return y


def _deg_body(row_hbm, ew_hbm, inv_hbm, idx_v, val_v, deg_sp):
    cid = lax.axis_index("c")
    tid = lax.axis_index("s")

    def zero(i, c):
        val_v[pl.ds(i * 16, 16)] = jnp.zeros((16,), jnp.float32)
        return c
    lax.fori_loop(0, RPT // 16, zero, 0)
    pltpu.sync_copy(val_v.at[pl.ds(0, RPT)], deg_sp.at[pl.ds(tid * RPT, RPT)])
    plsc.subcore_barrier()

    def win(wi, c):
        base = tid * EPT + wi * WD
        pltpu.sync_copy(row_hbm.at[pl.ds(base, WD)], idx_v)
        pltpu.sync_copy(ew_hbm.at[pl.ds(base, WD)], val_v)
        pltpu.sync_copy(val_v, deg_sp.at[idx_v], add=True)
        return c
    lax.fori_loop(0, NWD, win, 0)
    plsc.subcore_barrier()

    pltpu.sync_copy(deg_sp.at[pl.ds(tid * RPT, RPT)], val_v.at[pl.ds(0, RPT)])

    def inv(i, c):
        d = val_v[pl.ds(i * 16, 16)]
        d = jnp.where(d < 0.5, d + 1.0, d)
        val_v[pl.ds(i * 16, 16)] = 1.0 / d
        return c
    lax.fori_loop(0, RPT // 16, inv, 0)

    @pl.when(cid == 0)
    def _():
        pltpu.sync_copy(val_v.at[pl.ds(0, RPT)], inv_hbm.at[pl.ds(tid * RPT, RPT)])


def _deg_pass(row, ew):
    return pl.kernel(
        _deg_body,
        out_type=jax.ShapeDtypeStruct((NPAD,), jnp.float32),
        mesh=_mesh(),
        compiler_params=pltpu.CompilerParams(needs_layout_passes=False, use_tc_tiling_on_sc=False),
        scratch_types=[
            pltpu.VMEM((WD,), jnp.int32),
            pltpu.VMEM((WD,), jnp.float32),
            pltpu.VMEM_SHARED((NPAD,), jnp.float32),
        ],
    )(row, ew)


def _stats_reduce_and_consts(stats_sp, part_v, stat_v, s1s, s2s, tid):
    for c in range(4):
        part_v[0, pl.ds(c * 16, 16)] = s1s[c]
        part_v[1, pl.ds(c * 16, 16)] = s2s[c]
    pltpu.sync_copy(part_v.at[pl.ds(0, 1)], stats_sp.at[pl.ds(tid, 1)])
    pltpu.sync_copy(part_v.at[pl.ds(1, 1)], stats_sp.at[pl.ds(NT + tid, 1)])
    plsc.subcore_barrier()

    @pl.when(tid == 0)
    def _():
        pltpu.sync_copy(stats_sp.at[pl.ds(0, 2 * NT)], part_v.at[pl.ds(2, 2 * NT)])
        for c in range(4):
            def red(i, acc):
                a1, a2 = acc
                return (a1 + part_v[2 + i, pl.ds(c * 16, 16)],
                        a2 + part_v[2 + NT + i, pl.ds(c * 16, 16)])
            z = jnp.zeros((16,), jnp.float32)
            t1, t2 = lax.fori_loop(0, NT, red, (z, z))
            part_v[0, pl.ds(c * 16, 16)] = t1
            part_v[1, pl.ds(c * 16, 16)] = t2
        pltpu.sync_copy(part_v.at[pl.ds(0, 2)], stats_sp.at[pl.ds(2 * NT, 2)])
    plsc.subcore_barrier()
    pltpu.sync_copy(stats_sp.at[pl.ds(2 * NT, 2)], stat_v)


def _emb_body(x_hbm, tbl_hbm, gnp_hbm, h_hbm,
              tbl_v, xs_v, rows_v, part_v, stat_v, gn_v, stats_sp, sem):
    cid = lax.axis_index("c")
    tid = lax.axis_index("s")
    pltpu.sync_copy(tbl_hbm, tbl_v)
    pltpu.sync_copy(x_hbm.at[pl.ds(tid * RPT, RPT)], xs_v)
    roff = cid * 65

    def node(k, carry):
        s1s, s2s = carry
        f = jnp.where(tid * RPT + k < N, 1.0, 0.0)
        rowi = plsc.load_gather(xs_v, [jnp.full((16,), k, jnp.int32)]) + roff
        n1, n2 = [], []
        for c in range(4):
            cols = lax.iota(jnp.int32, 16) + c * 16
            r = plsc.load_gather(tbl_v, [rowi, cols])
            rows_v[k, pl.ds(c * 16, 16)] = r
            rf = r * f
            n1.append(s1s[c] + rf)
            n2.append(s2s[c] + rf * r)
        return tuple(n1), tuple(n2)

    z = jnp.zeros((16,), jnp.float32)
    s1s, s2s = lax.fori_loop(0, RPT, node, ((z,) * 4, (z,) * 4))

    _stats_reduce_and_consts(stats_sp, part_v, stat_v, s1s, s2s, tid)
    pltpu.sync_copy(gnp_hbm.at[pl.ds(cid * 8, 8)], gn_v)

    alphas, betas = [], []
    for c in range(4):
        cs = pl.ds(c * 16, 16)
        s1, s2 = stat_v[0, cs], stat_v[1, cs]
        w, b, ms = gn_v[0, cs], gn_v[1, cs], gn_v[2, cs]
        m = s1 * (1.0 / N)
        var = s2 * (1.0 / N) - m * m * ms * (2.0 - ms)
        a = w * _rsqrt16(var + EPS)
        alphas.append(a)
        betas.append(b - a * m * ms)

    def app(k, c):
        for ci in range(4):
            cs = pl.ds(ci * 16, 16)
            rows_v[k, cs] = rows_v[k, cs] * alphas[ci] + betas[ci]
        return c
    lax.fori_loop(0, RPT, app, 0)

    base = cid * N + tid * RPT

    @pl.when(tid < NT - 1)
    def _():
        pltpu.sync_copy(rows_v, h_hbm.at[pl.ds(base, RPT)])

    @pl.when(tid == NT - 1)
    def _():
        pltpu.sync_copy(rows_v.at[pl.ds(0, RVAL)], h_hbm.at[pl.ds(base, RVAL)])


def _emb_pass(x_pad, tbl2, gnp):
    return pl.kernel(
        _emb_body,
        out_type=jax.ShapeDtypeStruct((2 * N, HH), jnp.float32),
        mesh=_mesh(),
        compiler_params=pltpu.CompilerParams(needs_layout_passes=False, use_tc_tiling_on_sc=False),
        scratch_types=[
            pltpu.VMEM((130, HH), jnp.float32),
            pltpu.VMEM((RPT,), jnp.int32),
            pltpu.VMEM((RPT, HH), jnp.float32),
            pltpu.VMEM((2 + 2 * NT, HH), jnp.float32),
            pltpu.VMEM((2, HH), jnp.float32),
            pltpu.VMEM((8, HH), jnp.float32),
            pltpu.VMEM_SHARED((2 * NT + 2, HH), jnp.float32),
            pltpu.SemaphoreType.DMA,
        ],
    )(x_pad, tbl2, gnp)


def _spmm_body(h_in, row_hbm, col2_hbm, ew_hbm, inv_hbm, gnp_hbm, h_out,
               colA, colB, rowA, rowB, ewA, ewB, gA, gB, inv_v,
               part_v, stat_v, gn_v, acc_sp, stats_sp,
               siA, siB, sgA, sgB, ssA, ssB):
    cid = lax.axis_index("c")
    tid = lax.axis_index("s")
    HC = RPT // 2

    bufs = ((colA, rowA, ewA, gA, siA, sgA, ssA),
            (colB, rowB, ewB, gB, siB, sgB, ssB))

    def zrow(k, c):
        for ci in range(4):
            gA[k, pl.ds(ci * 16, 16)] = jnp.zeros((16,), jnp.float32)
        return c
    lax.fori_loop(0, WE, zrow, 0, unroll=4)
    pltpu.sync_copy(gA, acc_sp.at[pl.ds(tid * RPT, WE)])
    pltpu.sync_copy(gA.at[pl.ds(0, RPT - WE)],
                    acc_sp.at[pl.ds(tid * RPT + WE, RPT - WE)])
    plsc.subcore_barrier()

    ebase = tid * EPT

    def start_idx(b, w):
        col_v, row_v, ew_v, _, si, _, _ = bufs[b]
        pltpu.async_copy(col2_hbm.at[cid, pl.ds(ebase + w * WE, WE)], col_v, si)
        pltpu.async_copy(row_hbm.at[pl.ds(ebase + w * WE, WE)], row_v, si)
        pltpu.async_copy(ew_hbm.at[pl.ds(ebase + w * WE, WE)], ew_v, si)

    def wait_idx(b, w):
        col_v, row_v, ew_v, _, si, _, _ = bufs[b]
        pltpu.make_async_copy(col2_hbm.at[cid, pl.ds(ebase + w * WE, WE)], col_v, si).wait()
        pltpu.make_async_copy(row_hbm.at[pl.ds(ebase + w * WE, WE)], row_v, si).wait()
        pltpu.make_async_copy(ew_hbm.at[pl.ds(ebase + w * WE, WE)], ew_v, si).wait()

    def start_gather(b):
        pltpu.async_copy(h_in.at[bufs[b][0]], bufs[b][3], bufs[b][5])

    def wait_gather(b):
        pltpu.make_async_copy(h_in.at[bufs[b][0]], bufs[b][3], bufs[b][5]).wait()

    def scatter(b):
        pltpu.sync_copy(bufs[b][3], acc_sp.at[bufs[b][1]], add=True)

    def mul(b):
        _, _, ew_v, g_v, _, _, _ = bufs[b]

        def body(k, c):
            wk = plsc.load_gather(ew_v, [jnp.full((16,), k, jnp.int32)])
            for ci in range(4):
                cs = pl.ds(ci * 16, 16)
                g_v[k, cs] = g_v[k, cs] * wk
            return c
        lax.fori_loop(0, WE, body, 0, unroll=8)

    start_idx(0, 0)
    start_idx(1, 1)
    wait_idx(0, 0)
    start_gather(0)
    NH = NWE // 2

    def slots(h, c):
        i0 = 2 * h
        wait_gather(0)
        wait_idx(1, i0 + 1)
        start_gather(1)
        mul(0)
        scatter(0)

        @pl.when(h < NH - 1)
        def _():
            start_idx(0, i0 + 2)
        wait_gather(1)

        @pl.when(h < NH - 1)
        def _():
            wait_idx(0, i0 + 2)
            start_gather(0)
        mul(1)
        scatter(1)

        @pl.when(h < NH - 1)
        def _():
            start_idx(1, i0 + 3)
        return c
    lax.fori_loop(0, NH, slots, 0)
    plsc.subcore_barrier()

    r0 = tid * RPT
    pltpu.sync_copy(inv_hbm.at[pl.ds(r0, RPT)], inv_v)
    z = jnp.zeros((16,), jnp.float32)
    carry = ((z,) * 4, (z,) * 4)
    for sub in range(2):
        buf = bufs[sub][3]
        pltpu.sync_copy(acc_sp.at[pl.ds(r0 + sub * HC, HC)], buf.at[pl.ds(0, HC)])

        def srow(k, cr, _sub=sub, _buf=buf):
            s1s, s2s = cr
            f = jnp.where(r0 + _sub * HC + k < N, 1.0, 0.0)
            s = plsc.load_gather(
                inv_v, [jnp.full((16,), _sub * HC + k, jnp.int32)]) * f
            n1, n2 = [], []
            for ci in range(4):
                cs = pl.ds(ci * 16, 16)
                t = _buf[k, cs] * s
                _buf[k, cs] = t
                n1.append(s1s[ci] + t)
                n2.append(s2s[ci] + t * t)
            return tuple(n1), tuple(n2)
        carry = lax.fori_loop(0, HC, srow, carry, unroll=4)
    s1s, s2s = carry

    _stats_reduce_and_consts(stats_sp, part_v, stat_v, s1s, s2s, tid)
    pltpu.sync_copy(gnp_hbm.at[pl.ds(cid * 8, 8)], gn_v)

    alphas, betas, caps = [], [], []
    for c in range(4):
        cs = pl.ds(c * 16, 16)
        s1, s2 = stat_v[0, cs], stat_v[1, cs]
        w1, b1, ms1 = gn_v[0, cs], gn_v[1, cs], gn_v[2, cs]
        w2, b2, ms2 = gn_v[3, cs], gn_v[4, cs], gn_v[5, cs]
        cap = gn_v[6, cs]
        m = s1 * (1.0 / N)
        var1 = s2 * (1.0 / N) - m * m * ms1 * (2.0 - ms1)
        a1 = w1 * _rsqrt16(var1 + EPS)
        b1f = b1 - a1 * m * ms1
        m2 = a1 * m + b1f
        varh = s2 * (1.0 / N) - m * m
        mo = m2 * (1.0 - ms2)
        var2 = a1 * a1 * varh + mo * mo
        a2 = w2 * _rsqrt16(var2 + EPS)
        b2f = b2 - a2 * m2 * ms2
        alphas.append(a2 * a1)
        betas.append(a2 * b1f + b2f)
        caps.append(cap)

    for sub in range(2):
        buf = bufs[sub][3]

        def app(k, c, _buf=buf):
            for ci in range(4):
                cs = pl.ds(ci * 16, 16)
                t = _buf[k, cs] * alphas[ci] + betas[ci]
                _buf[k, cs] = jnp.maximum(t, caps[ci])
            return c
        lax.fori_loop(0, HC, app, 0, unroll=4)

    base = cid * N + r0

    @pl.when(tid < NT - 1)
    def _():
        pltpu.sync_copy(gA.at[pl.ds(0, HC)], h_out.at[pl.ds(base, HC)])
        pltpu.sync_copy(gB.at[pl.ds(0, HC)], h_out.at[pl.ds(base + HC, HC)])

    @pl.when(tid == NT - 1)
    def _():
        pltpu.sync_copy(gA.at[pl.ds(0, HC)], h_out.at[pl.ds(base, HC)])
        pltpu.sync_copy(gB.at[pl.ds(0, RVAL - HC)],
                        h_out.at[pl.ds(base + HC, RVAL - HC)])


def _spmm_pass(h, row, col2, ew, inv, gnp):
    return pl.kernel(
        _spmm_body,
        out_type=jax.ShapeDtypeStruct((2 * N, HH), jnp.float32),
        mesh=_mesh(),
        compiler_params=pltpu.CompilerParams(needs_layout_passes=False, use_tc_tiling_on_sc=False),
        scratch_types=[
            pltpu.VMEM((WE,), jnp.int32),
            pltpu.VMEM((WE,), jnp.int32),
            pltpu.VMEM((WE,), jnp.int32),
            pltpu.VMEM((WE,), jnp.int32),
            pltpu.VMEM((WE,), jnp.float32),
            pltpu.VMEM((WE,), jnp.float32),
            pltpu.VMEM((WE, HH), jnp.float32),
            pltpu.VMEM((WE, HH), jnp.float32),
            pltpu.VMEM((RPT,), jnp.float32),
            pltpu.VMEM((2 + 2 * NT, HH), jnp.float32),
            pltpu.VMEM((2, HH), jnp.float32),
            pltpu.VMEM((8, HH), jnp.float32),
            pltpu.VMEM_SHARED((NPAD, HH), jnp.float32),
            pltpu.VMEM_SHARED((2 * NT + 2, HH), jnp.float32),
            pltpu.SemaphoreType.DMA,
            pltpu.SemaphoreType.DMA,
            pltpu.SemaphoreType.DMA,
            pltpu.SemaphoreType.DMA,
            pltpu.SemaphoreType.DMA,
            pltpu.SemaphoreType.DMA,
        ],
    )(h, row, col2, ew, inv, gnp)


def kernel(x, edge_index, edge_weight, emb_table, emb_gn, conv_gns, layer_gns):
    row = edge_index[0]
    col = edge_index[1]
    x_pad = jnp.concatenate([x, jnp.zeros((NPAD - N,), jnp.int32)])
    tbl2 = jnp.concatenate([emb_table[:, :HH], emb_table[:, HH:]], axis=0)
    zrow = jnp.zeros((HH,), jnp.float32)
    gnp_emb = jnp.stack(
        [emb_gn["w"][:HH], emb_gn["b"][:HH], emb_gn["ms"][:HH]]
        + [zrow] * 5
        + [emb_gn["w"][HH:], emb_gn["b"][HH:], emb_gn["ms"][HH:]]
        + [zrow] * 5)

    col2 = jnp.stack([col, col + jnp.int32(N)])
    inv = _deg_pass(row, edge_weight)
    h = _emb_pass(x_pad, tbl2, gnp_emb)
    gnps = []
    for layer in range(3):
        p1, p2 = conv_gns[layer], layer_gns[layer]
        cap = jnp.full((HH,), 0.0 if layer < 2 else -3.4e38, jnp.float32)
        rows = []
        for half in (slice(0, HH), slice(HH, 2 * HH)):
            for p in (p1, p2):
                rows += [p["w"][half], p["b"][half], p["ms"][half]]
            rows += [cap, zrow]
        gnps.append(jnp.stack(rows))

    def step(h, gnp):
        return _spmm_pass(h, row, col2, edge_weight, inv, gnp), None

    h, _ = lax.scan(step, h, jnp.stack(gnps))
    return jnp.concatenate([h[:N], h[N:]], axis=1)


# speedup vs baseline: 8.6322x; 1.1034x over previous
import functools

import jax
import jax.numpy as jnp
from jax import lax
from jax.experimental import pallas as pl
from jax.experimental.pallas import tpu as pltpu
from jax.experimental.pallas import tpu_sc as plsc

N = 10000
NPAD = 10240
E = 320000
HH = 64
NT = 16
RPT = NPAD // NT
RVAL = N - 15 * RPT
EPT = E // NT
WD = 2000
WE = 400
NWD = EPT // WD
NWE = EPT // WE
EPS = 1e-6


def _mesh():
    return plsc.VectorSubcoreMesh(
        core_axis_name="c", subcore_axis_name="s", num_cores=2, num_subcores=16
    )


def _rsqrt16(v):
    i = lax.bitcast_convert_type(v, jnp.int32)
    i = jnp.int32(0x5F3759DF) - lax.shift_right_logical(i, 1)
    y = lax.bitcast_convert_type(i, jnp.float32)
    for _ in range(4):
        y = y * (1.5 - 0.5 * v * y * y)
    return y


def _deg_body(row_hbm, ew_hbm, inv_hbm, idx_v, val_v, deg_sp):
    cid = lax.axis_index("c")
    tid = lax.axis_index("s")

    def zero(i, c):
        val_v[pl.ds(i * 16, 16)] = jnp.zeros((16,), jnp.float32)
        return c
    lax.fori_loop(0, RPT // 16, zero, 0)
    pltpu.sync_copy(val_v.at[pl.ds(0, RPT)], deg_sp.at[pl.ds(tid * RPT, RPT)])
    plsc.subcore_barrier()

    def win(wi, c):
        base = tid * EPT + wi * WD
        pltpu.sync_copy(row_hbm.at[pl.ds(base, WD)], idx_v)
        pltpu.sync_copy(ew_hbm.at[pl.ds(base, WD)], val_v)
        pltpu.sync_copy(val_v, deg_sp.at[idx_v], add=True)
        return c
    lax.fori_loop(0, NWD, win, 0)
    plsc.subcore_barrier()

    pltpu.sync_copy(deg_sp.at[pl.ds(tid * RPT, RPT)], val_v.at[pl.ds(0, RPT)])

    def inv(i, c):
        d = val_v[pl.ds(i * 16, 16)]
        d = jnp.where(d < 0.5, d + 1.0, d)
        val_v[pl.ds(i * 16, 16)] = 1.0 / d
        return c
    lax.fori_loop(0, RPT // 16, inv, 0)

    @pl.when(cid == 0)
    def _():
        pltpu.sync_copy(val_v.at[pl.ds(0, RPT)], inv_hbm.at[pl.ds(tid * RPT, RPT)])


def _deg_pass(row, ew):
    return pl.kernel(
        _deg_body,
        out_type=jax.ShapeDtypeStruct((NPAD,), jnp.float32),
        mesh=_mesh(),
        compiler_params=pltpu.CompilerParams(needs_layout_passes=False, use_tc_tiling_on_sc=False),
        scratch_types=[
            pltpu.VMEM((WD,), jnp.int32),
            pltpu.VMEM((WD,), jnp.float32),
            pltpu.VMEM_SHARED((NPAD,), jnp.float32),
        ],
    )(row, ew)


def _stats_reduce_and_consts(stats_sp, part_v, stat_v, s1s, s2s, tid):
    for c in range(4):
        part_v[0, pl.ds(c * 16, 16)] = s1s[c]
        part_v[1, pl.ds(c * 16, 16)] = s2s[c]
    pltpu.sync_copy(part_v.at[pl.ds(0, 1)], stats_sp.at[pl.ds(tid, 1)])
    pltpu.sync_copy(part_v.at[pl.ds(1, 1)], stats_sp.at[pl.ds(NT + tid, 1)])
    plsc.subcore_barrier()

    @pl.when(tid == 0)
    def _():
        pltpu.sync_copy(stats_sp.at[pl.ds(0, 2 * NT)], part_v.at[pl.ds(2, 2 * NT)])
        for c in range(4):
            def red(i, acc):
                a1, a2 = acc
                return (a1 + part_v[2 + i, pl.ds(c * 16, 16)],
                        a2 + part_v[2 + NT + i, pl.ds(c * 16, 16)])
            z = jnp.zeros((16,), jnp.float32)
            t1, t2 = lax.fori_loop(0, NT, red, (z, z))
            part_v[0, pl.ds(c * 16, 16)] = t1
            part_v[1, pl.ds(c * 16, 16)] = t2
        pltpu.sync_copy(part_v.at[pl.ds(0, 2)], stats_sp.at[pl.ds(2 * NT, 2)])
    plsc.subcore_barrier()
    pltpu.sync_copy(stats_sp.at[pl.ds(2 * NT, 2)], stat_v)


def _emb_body(x_hbm, tbl_hbm, gnp_hbm, h_hbm,
              tbl_v, xs_v, rows_v, part_v, stat_v, gn_v, stats_sp, sem):
    cid = lax.axis_index("c")
    tid = lax.axis_index("s")
    pltpu.sync_copy(tbl_hbm, tbl_v)
    pltpu.sync_copy(x_hbm.at[pl.ds(tid * RPT, RPT)], xs_v)
    roff = cid * 65

    def node(k, carry):
        s1s, s2s = carry
        f = jnp.where(tid * RPT + k < N, 1.0, 0.0)
        rowi = plsc.load_gather(xs_v, [jnp.full((16,), k, jnp.int32)]) + roff
        n1, n2 = [], []
        for c in range(4):
            cols = lax.iota(jnp.int32, 16) + c * 16
            r = plsc.load_gather(tbl_v, [rowi, cols])
            rows_v[k, pl.ds(c * 16, 16)] = r
            rf = r * f
            n1.append(s1s[c] + rf)
            n2.append(s2s[c] + rf * r)
        return tuple(n1), tuple(n2)

    z = jnp.zeros((16,), jnp.float32)
    s1s, s2s = lax.fori_loop(0, RPT, node, ((z,) * 4, (z,) * 4))

    _stats_reduce_and_consts(stats_sp, part_v, stat_v, s1s, s2s, tid)
    pltpu.sync_copy(gnp_hbm.at[pl.ds(cid * 8, 8)], gn_v)

    alphas, betas = [], []
    for c in range(4):
        cs = pl.ds(c * 16, 16)
        s1, s2 = stat_v[0, cs], stat_v[1, cs]
        w, b, ms = gn_v[0, cs], gn_v[1, cs], gn_v[2, cs]
        m = s1 * (1.0 / N)
        var = s2 * (1.0 / N) - m * m * ms * (2.0 - ms)
        a = w * _rsqrt16(var + EPS)
        alphas.append(a)
        betas.append(b - a * m * ms)

    def app(k, c):
        for ci in range(4):
            cs = pl.ds(ci * 16, 16)
            rows_v[k, cs] = rows_v[k, cs] * alphas[ci] + betas[ci]
        return c
    lax.fori_loop(0, RPT, app, 0)

    base = cid * N + tid * RPT

    @pl.when(tid < NT - 1)
    def _():
        pltpu.sync_copy(rows_v, h_hbm.at[pl.ds(base, RPT)])

    @pl.when(tid == NT - 1)
    def _():
        pltpu.sync_copy(rows_v.at[pl.ds(0, RVAL)], h_hbm.at[pl.ds(base, RVAL)])


def _emb_pass(x_pad, tbl2, gnp):
    return pl.kernel(
        _emb_body,
        out_type=jax.ShapeDtypeStruct((2 * N, HH), jnp.float32),
        mesh=_mesh(),
        compiler_params=pltpu.CompilerParams(needs_layout_passes=False, use_tc_tiling_on_sc=False),
        scratch_types=[
            pltpu.VMEM((130, HH), jnp.float32),
            pltpu.VMEM((RPT,), jnp.int32),
            pltpu.VMEM((RPT, HH), jnp.float32),
            pltpu.VMEM((2 + 2 * NT, HH), jnp.float32),
            pltpu.VMEM((2, HH), jnp.float32),
            pltpu.VMEM((8, HH), jnp.float32),
            pltpu.VMEM_SHARED((2 * NT + 2, HH), jnp.float32),
            pltpu.SemaphoreType.DMA,
        ],
    )(x_pad, tbl2, gnp)


def _spmm_body(h_in, row_hbm, col2_hbm, ew_hbm, inv_hbm, gnp_hbm, h_out,
               colA, colB, rowA, rowB, rowSA, rowSB, ewA, ewB, gA, gB, inv_v,
               part_v, stat_v, gn_v, acc_sp, stats_sp,
               siA, siB, sgA, sgB, ssA, ssB):
    cid = lax.axis_index("c")
    tid = lax.axis_index("s")
    HC = RPT // 2

    bufs = ((colA, rowA, ewA, gA, siA, sgA, ssA, rowSA),
            (colB, rowB, ewB, gB, siB, sgB, ssB, rowSB))

    def zrow(k, c):
        for ci in range(4):
            gA[k, pl.ds(ci * 16, 16)] = jnp.zeros((16,), jnp.float32)
        return c
    lax.fori_loop(0, WE, zrow, 0, unroll=4)
    pltpu.sync_copy(gA, acc_sp.at[pl.ds(tid * RPT, WE)])
    pltpu.sync_copy(gA.at[pl.ds(0, RPT - WE)],
                    acc_sp.at[pl.ds(tid * RPT + WE, RPT - WE)])
    plsc.subcore_barrier()

    ebase = tid * EPT

    def start_idx(b, w):
        col_v, row_v, ew_v, _, si, _, _, _ = bufs[b]
        pltpu.async_copy(col2_hbm.at[cid, pl.ds(ebase + w * WE, WE)], col_v, si)
        pltpu.async_copy(row_hbm.at[pl.ds(ebase + w * WE, WE)], row_v, si)
        pltpu.async_copy(ew_hbm.at[pl.ds(ebase + w * WE, WE)], ew_v, si)

    def wait_idx(b, w):
        col_v, row_v, ew_v, _, si, _, _, _ = bufs[b]
        pltpu.make_async_copy(col2_hbm.at[cid, pl.ds(ebase + w * WE, WE)], col_v, si).wait()
        pltpu.make_async_copy(row_hbm.at[pl.ds(ebase + w * WE, WE)], row_v, si).wait()
        pltpu.make_async_copy(ew_hbm.at[pl.ds(ebase + w * WE, WE)], ew_v, si).wait()

    def start_gather(b):
        pltpu.async_copy(h_in.at[bufs[b][0]], bufs[b][3], bufs[b][5])

    def wait_gather(b):
        pltpu.make_async_copy(h_in.at[bufs[b][0]], bufs[b][3], bufs[b][5]).wait()

    def copy_rows(b):
        row_v, rowS = bufs[b][1], bufs[b][7]

        def body(i, c):
            cs = pl.ds(i * 16, 16)
            rowS[cs] = row_v[cs]
            return c
        lax.fori_loop(0, WE // 16, body, 0, unroll=5)

    def start_scatter(b):
        pltpu.async_copy(bufs[b][3], acc_sp.at[bufs[b][7]], bufs[b][6], add=True)

    def wait_scatter(b):
        pltpu.make_async_copy(bufs[b][3], acc_sp.at[bufs[b][7]], bufs[b][6]).wait()

    def mul(b):
        _, _, ew_v, g_v, _, _, _, _ = bufs[b]

        def body(k, c):
            wk = plsc.load_gather(ew_v, [jnp.full((16,), k, jnp.int32)])
            for ci in range(4):
                cs = pl.ds(ci * 16, 16)
                g_v[k, cs] = g_v[k, cs] * wk
            return c
        lax.fori_loop(0, WE, body, 0, unroll=8)

    start_idx(0, 0)
    start_idx(1, 1)
    wait_idx(0, 0)
    start_gather(0)
    NH = NWE // 2

    def slots(h, c):
        i0 = 2 * h
        @pl.when(h > 0)
        def _():
            wait_scatter(1)
        wait_idx(1, i0 + 1)
        start_gather(1)
        wait_gather(0)
        mul(0)
        copy_rows(0)
        start_scatter(0)

        @pl.when(h < NH - 1)
        def _():
            start_idx(0, i0 + 2)
        wait_scatter(0)

        @pl.when(h < NH - 1)
        def _():
            wait_idx(0, i0 + 2)
            start_gather(0)
        wait_gather(1)
        mul(1)
        copy_rows(1)
        start_scatter(1)

        @pl.when(h < NH - 1)
        def _():
            start_idx(1, i0 + 3)
        return c
    lax.fori_loop(0, NH, slots, 0)
    wait_scatter(1)
    plsc.subcore_barrier()

    r0 = tid * RPT
    pltpu.sync_copy(inv_hbm.at[pl.ds(r0, RPT)], inv_v)
    z = jnp.zeros((16,), jnp.float32)
    carry = ((z,) * 4, (z,) * 4)
    for sub in range(2):
        buf = bufs[sub][3]
        pltpu.sync_copy(acc_sp.at[pl.ds(r0 + sub * HC, HC)], buf.at[pl.ds(0, HC)])

        def srow(k, cr, _sub=sub, _buf=buf):
            s1s, s2s = cr
            f = jnp.where(r0 + _sub * HC + k < N, 1.0, 0.0)
            s = plsc.load_gather(
                inv_v, [jnp.full((16,), _sub * HC + k, jnp.int32)]) * f
            n1, n2 = [], []
            for ci in range(4):
                cs = pl.ds(ci * 16, 16)
                t = _buf[k, cs] * s
                _buf[k, cs] = t
                n1.append(s1s[ci] + t)
                n2.append(s2s[ci] + t * t)
            return tuple(n1), tuple(n2)
        carry = lax.fori_loop(0, HC, srow, carry, unroll=4)
    s1s, s2s = carry

    _stats_reduce_and_consts(stats_sp, part_v, stat_v, s1s, s2s, tid)
    pltpu.sync_copy(gnp_hbm.at[pl.ds(cid * 8, 8)], gn_v)

    alphas, betas, caps = [], [], []
    for c in range(4):
        cs = pl.ds(c * 16, 16)
        s1, s2 = stat_v[0, cs], stat_v[1, cs]
        w1, b1, ms1 = gn_v[0, cs], gn_v[1, cs], gn_v[2, cs]
        w2, b2, ms2 = gn_v[3, cs], gn_v[4, cs], gn_v[5, cs]
        cap = gn_v[6, cs]
        m = s1 * (1.0 / N)
        var1 = s2 * (1.0 / N) - m * m * ms1 * (2.0 - ms1)
        a1 = w1 * _rsqrt16(var1 + EPS)
        b1f = b1 - a1 * m * ms1
        m2 = a1 * m + b1f
        varh = s2 * (1.0 / N) - m * m
        mo = m2 * (1.0 - ms2)
        var2 = a1 * a1 * varh + mo * mo
        a2 = w2 * _rsqrt16(var2 + EPS)
        b2f = b2 - a2 * m2 * ms2
        alphas.append(a2 * a1)
        betas.append(a2 * b1f + b2f)
        caps.append(cap)

    for sub in range(2):
        buf = bufs[sub][3]

        def app(k, c, _buf=buf):
            for ci in range(4):
                cs = pl.ds(ci * 16, 16)
                t = _buf[k, cs] * alphas[ci] + betas[ci]
                _buf[k, cs] = jnp.maximum(t, caps[ci])
            return c
        lax.fori_loop(0, HC, app, 0, unroll=4)

    base = cid * N + r0

    @pl.when(tid < NT - 1)
    def _():
        pltpu.sync_copy(gA.at[pl.ds(0, HC)], h_out.at[pl.ds(base, HC)])
        pltpu.sync_copy(gB.at[pl.ds(0, HC)], h_out.at[pl.ds(base + HC, HC)])

    @pl.when(tid == NT - 1)
    def _():
        pltpu.sync_copy(gA.at[pl.ds(0, HC)], h_out.at[pl.ds(base, HC)])
        pltpu.sync_copy(gB.at[pl.ds(0, RVAL - HC)],
                        h_out.at[pl.ds(base + HC, RVAL - HC)])


def _spmm_pass(h, row, col2, ew, inv, gnp):
    return pl.kernel(
        _spmm_body,
        out_type=jax.ShapeDtypeStruct((2 * N, HH), jnp.float32),
        mesh=_mesh(),
        compiler_params=pltpu.CompilerParams(needs_layout_passes=False, use_tc_tiling_on_sc=False),
        scratch_types=[
            pltpu.VMEM((WE,), jnp.int32),
            pltpu.VMEM((WE,), jnp.int32),
            pltpu.VMEM((WE,), jnp.int32),
            pltpu.VMEM((WE,), jnp.int32),
            pltpu.VMEM((WE,), jnp.int32),
            pltpu.VMEM((WE,), jnp.int32),
            pltpu.VMEM((WE,), jnp.float32),
            pltpu.VMEM((WE,), jnp.float32),
            pltpu.VMEM((WE, HH), jnp.float32),
            pltpu.VMEM((WE, HH), jnp.float32),
            pltpu.VMEM((RPT,), jnp.float32),
            pltpu.VMEM((2 + 2 * NT, HH), jnp.float32),
            pltpu.VMEM((2, HH), jnp.float32),
            pltpu.VMEM((8, HH), jnp.float32),
            pltpu.VMEM_SHARED((NPAD, HH), jnp.float32),
            pltpu.VMEM_SHARED((2 * NT + 2, HH), jnp.float32),
            pltpu.SemaphoreType.DMA,
            pltpu.SemaphoreType.DMA,
            pltpu.SemaphoreType.DMA,
            pltpu.SemaphoreType.DMA,
            pltpu.SemaphoreType.DMA,
            pltpu.SemaphoreType.DMA,
        ],
    )(h, row, col2, ew, inv, gnp)


def kernel(x, edge_index, edge_weight, emb_table, emb_gn, conv_gns, layer_gns):
    row = edge_index[0]
    col = edge_index[1]
    x_pad = jnp.concatenate([x, jnp.zeros((NPAD - N,), jnp.int32)])
    tbl2 = jnp.concatenate([emb_table[:, :HH], emb_table[:, HH:]], axis=0)
    zrow = jnp.zeros((HH,), jnp.float32)
    gnp_emb = jnp.stack(
        [emb_gn["w"][:HH], emb_gn["b"][:HH], emb_gn["ms"][:HH]]
        + [zrow] * 5
        + [emb_gn["w"][HH:], emb_gn["b"][HH:], emb_gn["ms"][HH:]]
        + [zrow] * 5)

    col2 = jnp.stack([col, col + jnp.int32(N)])
    inv = _deg_pass(row, edge_weight)
    h = _emb_pass(x_pad, tbl2, gnp_emb)
    gnps = []
    for layer in range(3):
        p1, p2 = conv_gns[layer], layer_gns[layer]
        cap = jnp.full((HH,), 0.0 if layer < 2 else -3.4e38, jnp.float32)
        rows = []
        for half in (slice(0, HH), slice(HH, 2 * HH)):
            for p in (p1, p2):
                rows += [p["w"][half], p["b"][half], p["ms"][half]]
            rows += [cap, zrow]
        gnps.append(jnp.stack(rows))

    def step(h, gnp):
        return _spmm_pass(h, row, col2, edge_weight, inv, gnp), None

    h, _ = lax.scan(step, h, jnp.stack(gnps))
    return jnp.concatenate([h[:N], h[N:]], axis=1)


# speedup vs baseline: 10.7834x; 1.2492x over previous
import functools

import jax
import jax.numpy as jnp
from jax import lax
from jax.experimental import pallas as pl
from jax.experimental.pallas import tpu as pltpu
from jax.experimental.pallas import tpu_sc as plsc

N = 10000
NPAD = 10240
E = 320000
HH = 64
NT = 16
RPT = NPAD // NT
RVAL = N - 15 * RPT
EPT = E // NT
WD = 2000
WE = 400
NWD = EPT // WD
NWE = EPT // WE
EPS = 1e-6


def _mesh():
    return plsc.VectorSubcoreMesh(
        core_axis_name="c", subcore_axis_name="s", num_cores=2, num_subcores=16
    )


def _rsqrt16(v):
    i = lax.bitcast_convert_type(v, jnp.int32)
    i = jnp.int32(0x5F3759DF) - lax.shift_right_logical(i, 1)
    y = lax.bitcast_convert_type(i, jnp.float32)
    for _ in range(4):
        y = y * (1.5 - 0.5 * v * y * y)
    return y


def _deg_body(row_hbm, ew_hbm, inv_hbm, idx_v, val_v, deg_sp):
    cid = lax.axis_index("c")
    tid = lax.axis_index("s")

    def zero(i, c):
        val_v[pl.ds(i * 16, 16)] = jnp.zeros((16,), jnp.float32)
        return c
    lax.fori_loop(0, RPT // 16, zero, 0)
    pltpu.sync_copy(val_v.at[pl.ds(0, RPT)], deg_sp.at[pl.ds(tid * RPT, RPT)])
    plsc.subcore_barrier()

    def win(wi, c):
        base = tid * EPT + wi * WD
        pltpu.sync_copy(row_hbm.at[pl.ds(base, WD)], idx_v)
        pltpu.sync_copy(ew_hbm.at[pl.ds(base, WD)], val_v)
        pltpu.sync_copy(val_v, deg_sp.at[idx_v], add=True)
        return c
    lax.fori_loop(0, NWD, win, 0)
    plsc.subcore_barrier()

    pltpu.sync_copy(deg_sp.at[pl.ds(tid * RPT, RPT)], val_v.at[pl.ds(0, RPT)])

    def inv(i, c):
        d = val_v[pl.ds(i * 16, 16)]
        d = jnp.where(d < 0.5, d + 1.0, d)
        val_v[pl.ds(i * 16, 16)] = 1.0 / d
        return c
    lax.fori_loop(0, RPT // 16, inv, 0)

    @pl.when(cid == 0)
    def _():
        pltpu.sync_copy(val_v.at[pl.ds(0, RPT)], inv_hbm.at[pl.ds(tid * RPT, RPT)])


def _deg_pass(row, ew):
    return pl.kernel(
        _deg_body,
        out_type=jax.ShapeDtypeStruct((NPAD,), jnp.float32),
        mesh=_mesh(),
        compiler_params=pltpu.CompilerParams(needs_layout_passes=False, use_tc_tiling_on_sc=False),
        scratch_types=[
            pltpu.VMEM((WD,), jnp.int32),
            pltpu.VMEM((WD,), jnp.float32),
            pltpu.VMEM_SHARED((NPAD,), jnp.float32),
        ],
    )(row, ew)


def _stats_reduce_and_consts(stats_sp, part_v, stat_v, s1s, s2s, tid):
    for c in range(4):
        part_v[0, pl.ds(c * 16, 16)] = s1s[c]
        part_v[1, pl.ds(c * 16, 16)] = s2s[c]
    pltpu.sync_copy(part_v.at[pl.ds(0, 1)], stats_sp.at[pl.ds(tid, 1)])
    pltpu.sync_copy(part_v.at[pl.ds(1, 1)], stats_sp.at[pl.ds(NT + tid, 1)])
    plsc.subcore_barrier()

    @pl.when(tid == 0)
    def _():
        pltpu.sync_copy(stats_sp.at[pl.ds(0, 2 * NT)], part_v.at[pl.ds(2, 2 * NT)])
        for c in range(4):
            def red(i, acc):
                a1, a2 = acc
                return (a1 + part_v[2 + i, pl.ds(c * 16, 16)],
                        a2 + part_v[2 + NT + i, pl.ds(c * 16, 16)])
            z = jnp.zeros((16,), jnp.float32)
            t1, t2 = lax.fori_loop(0, NT, red, (z, z))
            part_v[0, pl.ds(c * 16, 16)] = t1
            part_v[1, pl.ds(c * 16, 16)] = t2
        pltpu.sync_copy(part_v.at[pl.ds(0, 2)], stats_sp.at[pl.ds(2 * NT, 2)])
    plsc.subcore_barrier()
    pltpu.sync_copy(stats_sp.at[pl.ds(2 * NT, 2)], stat_v)


def _emb_body(x_hbm, tbl_hbm, gnp_hbm, h_hbm,
              tbl_v, xs_v, rows_v, part_v, stat_v, gn_v, stats_sp, sem):
    cid = lax.axis_index("c")
    tid = lax.axis_index("s")
    pltpu.sync_copy(tbl_hbm, tbl_v)
    pltpu.sync_copy(x_hbm.at[pl.ds(tid * RPT, RPT)], xs_v)
    roff = cid * 65

    def node(k, carry):
        s1s, s2s = carry
        f = jnp.where(tid * RPT + k < N, 1.0, 0.0)
        rowi = plsc.load_gather(xs_v, [jnp.full((16,), k, jnp.int32)]) + roff
        n1, n2 = [], []
        for c in range(4):
            cols = lax.iota(jnp.int32, 16) + c * 16
            r = plsc.load_gather(tbl_v, [rowi, cols])
            rows_v[k, pl.ds(c * 16, 16)] = r
            rf = r * f
            n1.append(s1s[c] + rf)
            n2.append(s2s[c] + rf * r)
        return tuple(n1), tuple(n2)

    z = jnp.zeros((16,), jnp.float32)
    s1s, s2s = lax.fori_loop(0, RPT, node, ((z,) * 4, (z,) * 4))

    _stats_reduce_and_consts(stats_sp, part_v, stat_v, s1s, s2s, tid)
    pltpu.sync_copy(gnp_hbm.at[pl.ds(cid * 8, 8)], gn_v)

    alphas, betas = [], []
    for c in range(4):
        cs = pl.ds(c * 16, 16)
        s1, s2 = stat_v[0, cs], stat_v[1, cs]
        w, b, ms = gn_v[0, cs], gn_v[1, cs], gn_v[2, cs]
        m = s1 * (1.0 / N)
        var = s2 * (1.0 / N) - m * m * ms * (2.0 - ms)
        a = w * _rsqrt16(var + EPS)
        alphas.append(a)
        betas.append(b - a * m * ms)

    def app(k, c):
        for ci in range(4):
            cs = pl.ds(ci * 16, 16)
            rows_v[k, cs] = rows_v[k, cs] * alphas[ci] + betas[ci]
        return c
    lax.fori_loop(0, RPT, app, 0)

    base = cid * N + tid * RPT

    @pl.when(tid < NT - 1)
    def _():
        pltpu.sync_copy(rows_v, h_hbm.at[pl.ds(base, RPT)])

    @pl.when(tid == NT - 1)
    def _():
        pltpu.sync_copy(rows_v.at[pl.ds(0, RVAL)], h_hbm.at[pl.ds(base, RVAL)])


def _emb_pass(x_pad, tbl2, gnp):
    return pl.kernel(
        _emb_body,
        out_type=jax.ShapeDtypeStruct((2 * N, HH), jnp.float32),
        mesh=_mesh(),
        compiler_params=pltpu.CompilerParams(needs_layout_passes=False, use_tc_tiling_on_sc=False),
        scratch_types=[
            pltpu.VMEM((130, HH), jnp.float32),
            pltpu.VMEM((RPT,), jnp.int32),
            pltpu.VMEM((RPT, HH), jnp.float32),
            pltpu.VMEM((2 + 2 * NT, HH), jnp.float32),
            pltpu.VMEM((2, HH), jnp.float32),
            pltpu.VMEM((8, HH), jnp.float32),
            pltpu.VMEM_SHARED((2 * NT + 2, HH), jnp.float32),
            pltpu.SemaphoreType.DMA,
        ],
    )(x_pad, tbl2, gnp)


def _spmm_body(h_in, ed_hbm, inv_hbm, gnp_hbm, h_out,
               eA, eB, rowSA, rowSB, gA, gB, inv_v,
               part_v, stat_v, gn_v, acc_sp, stats_sp,
               siA, siB, sgA, sgB, ssA, ssB):
    cid = lax.axis_index("c")
    tid = lax.axis_index("s")
    HC = RPT // 2

    bufs = ((eA, gA, siA, sgA, ssA, rowSA),
            (eB, gB, siB, sgB, ssB, rowSB))

    def zrow(k, c):
        for ci in range(4):
            gA[k, pl.ds(ci * 16, 16)] = jnp.zeros((16,), jnp.float32)
        return c
    lax.fori_loop(0, WE, zrow, 0, unroll=4)
    pltpu.sync_copy(gA, acc_sp.at[pl.ds(tid * RPT, WE)])
    pltpu.sync_copy(gA.at[pl.ds(0, RPT - WE)],
                    acc_sp.at[pl.ds(tid * RPT + WE, RPT - WE)])
    plsc.subcore_barrier()

    wbase = tid * NWE

    def start_idx(b, w):
        e_v, _, si, _, _, _ = bufs[b]
        pltpu.async_copy(ed_hbm.at[cid, wbase + w], e_v, si)

    def wait_idx(b, w):
        e_v, _, si, _, _, _ = bufs[b]
        pltpu.make_async_copy(ed_hbm.at[cid, wbase + w], e_v, si).wait()

    def start_gather(b):
        pltpu.async_copy(h_in.at[bufs[b][0].at[0]], bufs[b][1], bufs[b][3])

    def wait_gather(b):
        pltpu.make_async_copy(h_in.at[bufs[b][0].at[0]], bufs[b][1], bufs[b][3]).wait()

    def copy_rows(b):
        e_v, rowS = bufs[b][0], bufs[b][5]

        def body(i, c):
            cs = pl.ds(i * 16, 16)
            rowS[cs] = e_v[1, cs]
            return c
        lax.fori_loop(0, WE // 16, body, 0, unroll=5)

    def start_scatter(b):
        pltpu.async_copy(bufs[b][1], acc_sp.at[bufs[b][5]], bufs[b][4], add=True)

    def wait_scatter(b):
        pltpu.make_async_copy(bufs[b][1], acc_sp.at[bufs[b][5]], bufs[b][4]).wait()

    def mul(b):
        e_v, g_v = bufs[b][0], bufs[b][1]

        def body(g, c):
            w16 = plsc.bitcast(e_v[2, pl.ds(g * 16, 16)], jnp.float32)
            for j in range(16):
                wk = jnp.full((16,), w16[j], jnp.float32)
                k = g * 16 + j
                for ci in range(4):
                    cs = pl.ds(ci * 16, 16)
                    g_v[k, cs] = g_v[k, cs] * wk
            return c
        lax.fori_loop(0, WE // 16, body, 0, unroll=2)

    start_idx(0, 0)
    start_idx(1, 1)
    wait_idx(0, 0)
    start_gather(0)
    NH = NWE // 2

    def slots(h, c):
        i0 = 2 * h
        @pl.when(h > 0)
        def _():
            wait_scatter(1)
        wait_idx(1, i0 + 1)
        start_gather(1)
        wait_gather(0)
        mul(0)
        copy_rows(0)
        start_scatter(0)

        @pl.when(h < NH - 1)
        def _():
            start_idx(0, i0 + 2)
        wait_scatter(0)

        @pl.when(h < NH - 1)
        def _():
            wait_idx(0, i0 + 2)
            start_gather(0)
        wait_gather(1)
        mul(1)
        copy_rows(1)
        start_scatter(1)

        @pl.when(h < NH - 1)
        def _():
            start_idx(1, i0 + 3)
        return c
    lax.fori_loop(0, NH, slots, 0)
    wait_scatter(1)
    plsc.subcore_barrier()

    r0 = tid * RPT
    pltpu.sync_copy(inv_hbm.at[pl.ds(r0, RPT)], inv_v)
    z = jnp.zeros((16,), jnp.float32)
    carry = ((z,) * 4, (z,) * 4)
    for sub in range(2):
        buf = bufs[sub][1]
        pltpu.sync_copy(acc_sp.at[pl.ds(r0 + sub * HC, HC)], buf.at[pl.ds(0, HC)])

        def srow(k, cr, _sub=sub, _buf=buf):
            s1s, s2s = cr
            f = jnp.where(r0 + _sub * HC + k < N, 1.0, 0.0)
            s = plsc.load_gather(
                inv_v, [jnp.full((16,), _sub * HC + k, jnp.int32)]) * f
            n1, n2 = [], []
            for ci in range(4):
                cs = pl.ds(ci * 16, 16)
                t = _buf[k, cs] * s
                _buf[k, cs] = t
                n1.append(s1s[ci] + t)
                n2.append(s2s[ci] + t * t)
            return tuple(n1), tuple(n2)
        carry = lax.fori_loop(0, HC, srow, carry, unroll=4)
    s1s, s2s = carry

    _stats_reduce_and_consts(stats_sp, part_v, stat_v, s1s, s2s, tid)
    pltpu.sync_copy(gnp_hbm.at[pl.ds(cid * 8, 8)], gn_v)

    alphas, betas, caps = [], [], []
    for c in range(4):
        cs = pl.ds(c * 16, 16)
        s1, s2 = stat_v[0, cs], stat_v[1, cs]
        w1, b1, ms1 = gn_v[0, cs], gn_v[1, cs], gn_v[2, cs]
        w2, b2, ms2 = gn_v[3, cs], gn_v[4, cs], gn_v[5, cs]
        cap = gn_v[6, cs]
        m = s1 * (1.0 / N)
        var1 = s2 * (1.0 / N) - m * m * ms1 * (2.0 - ms1)
        a1 = w1 * _rsqrt16(var1 + EPS)
        b1f = b1 - a1 * m * ms1
        m2 = a1 * m + b1f
        varh = s2 * (1.0 / N) - m * m
        mo = m2 * (1.0 - ms2)
        var2 = a1 * a1 * varh + mo * mo
        a2 = w2 * _rsqrt16(var2 + EPS)
        b2f = b2 - a2 * m2 * ms2
        alphas.append(a2 * a1)
        betas.append(a2 * b1f + b2f)
        caps.append(cap)

    for sub in range(2):
        buf = bufs[sub][1]

        def app(k, c, _buf=buf):
            for ci in range(4):
                cs = pl.ds(ci * 16, 16)
                t = _buf[k, cs] * alphas[ci] + betas[ci]
                _buf[k, cs] = jnp.maximum(t, caps[ci])
            return c
        lax.fori_loop(0, HC, app, 0, unroll=4)

    base = cid * N + r0

    @pl.when(tid < NT - 1)
    def _():
        pltpu.sync_copy(gA.at[pl.ds(0, HC)], h_out.at[pl.ds(base, HC)])
        pltpu.sync_copy(gB.at[pl.ds(0, HC)], h_out.at[pl.ds(base + HC, HC)])

    @pl.when(tid == NT - 1)
    def _():
        pltpu.sync_copy(gA.at[pl.ds(0, HC)], h_out.at[pl.ds(base, HC)])
        pltpu.sync_copy(gB.at[pl.ds(0, RVAL - HC)],
                        h_out.at[pl.ds(base + HC, RVAL - HC)])


def _spmm_pass(h, ed, inv, gnp):
    return pl.kernel(
        _spmm_body,
        out_type=jax.ShapeDtypeStruct((2 * N, HH), jnp.float32),
        mesh=_mesh(),
        compiler_params=pltpu.CompilerParams(needs_layout_passes=False, use_tc_tiling_on_sc=False),
        scratch_types=[
            pltpu.VMEM((3, WE), jnp.int32),
            pltpu.VMEM((3, WE), jnp.int32),
            pltpu.VMEM((WE,), jnp.int32),
            pltpu.VMEM((WE,), jnp.int32),
            pltpu.VMEM((WE, HH), jnp.float32),
            pltpu.VMEM((WE, HH), jnp.float32),
            pltpu.VMEM((RPT,), jnp.float32),
            pltpu.VMEM((2 + 2 * NT, HH), jnp.float32),
            pltpu.VMEM((2, HH), jnp.float32),
            pltpu.VMEM((8, HH), jnp.float32),
            pltpu.VMEM_SHARED((NPAD, HH), jnp.float32),
            pltpu.VMEM_SHARED((2 * NT + 2, HH), jnp.float32),
            pltpu.SemaphoreType.DMA,
            pltpu.SemaphoreType.DMA,
            pltpu.SemaphoreType.DMA,
            pltpu.SemaphoreType.DMA,
            pltpu.SemaphoreType.DMA,
            pltpu.SemaphoreType.DMA,
        ],
    )(h, ed, inv, gnp)


def kernel(x, edge_index, edge_weight, emb_table, emb_gn, conv_gns, layer_gns):
    row = edge_index[0]
    col = edge_index[1]
    x_pad = jnp.concatenate([x, jnp.zeros((NPAD - N,), jnp.int32)])
    tbl2 = jnp.concatenate([emb_table[:, :HH], emb_table[:, HH:]], axis=0)
    zrow = jnp.zeros((HH,), jnp.float32)
    gnp_emb = jnp.stack(
        [emb_gn["w"][:HH], emb_gn["b"][:HH], emb_gn["ms"][:HH]]
        + [zrow] * 5
        + [emb_gn["w"][HH:], emb_gn["b"][HH:], emb_gn["ms"][HH:]]
        + [zrow] * 5)

    ewbits = lax.bitcast_convert_type(edge_weight, jnp.int32)
    e0 = jnp.stack([col, row, ewbits])
    e1 = jnp.stack([col + jnp.int32(N), row, ewbits])
    ed = jnp.stack([e0, e1]).reshape(2, 3, E // WE, WE).transpose(0, 2, 1, 3)
    inv = _deg_pass(row, edge_weight)
    h = _emb_pass(x_pad, tbl2, gnp_emb)
    gnps = []
    for layer in range(3):
        p1, p2 = conv_gns[layer], layer_gns[layer]
        cap = jnp.full((HH,), 0.0 if layer < 2 else -3.4e38, jnp.float32)
        rows = []
        for half in (slice(0, HH), slice(HH, 2 * HH)):
            for p in (p1, p2):
                rows += [p["w"][half], p["b"][half], p["ms"][half]]
            rows += [cap, zrow]
        gnps.append(jnp.stack(rows))

    def step(h, gnp):
        return _spmm_pass(h, ed, inv, gnp), None

    h, _ = lax.scan(step, h, jnp.stack(gnps))
    return jnp.concatenate([h[:N], h[N:]], axis=1)


# speedup vs baseline: 12.9538x; 1.2013x over previous
import jax
import jax.numpy as jnp
from jax import lax
from jax.experimental import pallas as pl
from jax.experimental.pallas import tpu as pltpu
from jax.experimental.pallas import tpu_sc as plsc

N = 10000
NPAD = 10240
E = 320000
H = 128
HH = 64
NT = 16
RPT = NPAD // NT
RVAL = N - 15 * RPT
EPT = E // NT
WD = 2000
WE = 400
NWD = EPT // WD
NWE = EPT // WE
EPS = 1e-6


def _mesh():
    return plsc.VectorSubcoreMesh(
        core_axis_name="c", subcore_axis_name="s", num_cores=2, num_subcores=16
    )


def _rsqrt16(v):
    i = lax.bitcast_convert_type(v, jnp.int32)
    i = jnp.int32(0x5F3759DF) - lax.shift_right_logical(i, 1)
    y = lax.bitcast_convert_type(i, jnp.float32)
    for _ in range(4):
        y = y * (1.5 - 0.5 * v * y * y)
    return y


def _stats_reduce(stats_sp, part_v, stat_v, s1s, s2s, tid):
    for c in range(4):
        part_v[0, pl.ds(c * 16, 16)] = s1s[c]
        part_v[1, pl.ds(c * 16, 16)] = s2s[c]
    pltpu.sync_copy(part_v.at[pl.ds(0, 1)], stats_sp.at[pl.ds(tid, 1)])
    pltpu.sync_copy(part_v.at[pl.ds(1, 1)], stats_sp.at[pl.ds(NT + tid, 1)])
    plsc.subcore_barrier()

    @pl.when(tid == 0)
    def _():
        pltpu.sync_copy(stats_sp.at[pl.ds(0, 2 * NT)], part_v.at[pl.ds(2, 2 * NT)])
        for c in range(4):
            def red(i, acc):
                a1, a2 = acc
                return (a1 + part_v[2 + i, pl.ds(c * 16, 16)],
                        a2 + part_v[2 + NT + i, pl.ds(c * 16, 16)])
            z = jnp.zeros((16,), jnp.float32)
            t1, t2 = lax.fori_loop(0, NT, red, (z, z))
            part_v[0, pl.ds(c * 16, 16)] = t1
            part_v[1, pl.ds(c * 16, 16)] = t2
        pltpu.sync_copy(part_v.at[pl.ds(0, 2)], stats_sp.at[pl.ds(2 * NT, 2)])
    plsc.subcore_barrier()
    pltpu.sync_copy(stats_sp.at[pl.ds(2 * NT, 2)], stat_v)


def _fused_body(x_hbm, tbl_hbm, row_hbm, ew_hbm, ed_hbm, gnp_hbm,
                out_hbm, h1_hbm, h2_hbm,
                eA, eB, rowSA, rowSB, gA, gB, rowD, ewD,
                inv_v, xs_v, tbl_v, part_v, stat_v, gn_v,
                acc_sp, deg_sp, stats_sp,
                siA, siB, sgA, sgB, ssA, ssB):
    cid = lax.axis_index("c")
    tid = lax.axis_index("s")
    HC = RPT // 2
    r0 = tid * RPT

    bufs = ((eA, gA, siA, sgA, ssA, rowSA),
            (eB, gB, siB, sgB, ssB, rowSB))

    def zero16(i, c):
        inv_v[pl.ds(i * 16, 16)] = jnp.zeros((16,), jnp.float32)
        return c
    lax.fori_loop(0, RPT // 16, zero16, 0, unroll=4)
    pltpu.sync_copy(inv_v, deg_sp.at[pl.ds(r0, RPT)])
    plsc.subcore_barrier()

    def dwin(wi, c):
        dbase = tid * EPT + wi * WD
        pltpu.sync_copy(row_hbm.at[pl.ds(dbase, WD)], rowD)
        pltpu.sync_copy(ew_hbm.at[pl.ds(dbase, WD)], ewD)
        pltpu.sync_copy(ewD, deg_sp.at[rowD], add=True)
        return c
    lax.fori_loop(0, NWD, dwin, 0)
    plsc.subcore_barrier()

    pltpu.sync_copy(deg_sp.at[pl.ds(r0, RPT)], inv_v)

    def dinv(i, c):
        d = inv_v[pl.ds(i * 16, 16)]
        d = jnp.where(d < 0.5, d + 1.0, d)
        inv_v[pl.ds(i * 16, 16)] = 1.0 / d
        return c
    lax.fori_loop(0, RPT // 16, dinv, 0, unroll=4)
    pltpu.sync_copy(inv_v, deg_sp.at[pl.ds(r0, RPT)])

    pltpu.sync_copy(tbl_hbm, tbl_v)
    pltpu.sync_copy(x_hbm.at[pl.ds(r0, RPT)], xs_v)
    roff = cid * 65

    z = jnp.zeros((16,), jnp.float32)
    carry = ((z,) * 4, (z,) * 4)
    for sub in range(2):
        buf = bufs[sub][1]

        def enode(k, cr, _sub=sub, _buf=buf):
            s1s, s2s = cr
            f = jnp.where(r0 + _sub * HC + k < N, 1.0, 0.0)
            rowi = plsc.load_gather(
                xs_v, [jnp.full((16,), _sub * HC + k, jnp.int32)]) + roff
            n1, n2 = [], []
            for c in range(4):
                cols = lax.iota(jnp.int32, 16) + c * 16
                r = plsc.load_gather(tbl_v, [rowi, cols])
                _buf[k, pl.ds(c * 16, 16)] = r
                rf = r * f
                n1.append(s1s[c] + rf)
                n2.append(s2s[c] + rf * r)
            return tuple(n1), tuple(n2)
        carry = lax.fori_loop(0, HC, enode, carry, unroll=2)
    s1s, s2s = carry

    _stats_reduce(stats_sp, part_v, stat_v, s1s, s2s, tid)
    pltpu.sync_copy(gnp_hbm.at[pl.ds(cid * 8, 8)], gn_v)

    alphas, betas = [], []
    for c in range(4):
        cs = pl.ds(c * 16, 16)
        s1, s2 = stat_v[0, cs], stat_v[1, cs]
        w, b, ms = gn_v[0, cs], gn_v[1, cs], gn_v[2, cs]
        m = s1 * (1.0 / N)
        var = s2 * (1.0 / N) - m * m * ms * (2.0 - ms)
        a = w * _rsqrt16(var + EPS)
        alphas.append(a)
        betas.append(b - a * m * ms)

    for sub in range(2):
        buf = bufs[sub][1]

        def eapp(k, c, _buf=buf):
            for ci in range(4):
                cs = pl.ds(ci * 16, 16)
                _buf[k, cs] = _buf[k, cs] * alphas[ci] + betas[ci]
            return c
        lax.fori_loop(0, HC, eapp, 0, unroll=2)

    ebase = cid * N + r0

    @pl.when(tid < NT - 1)
    def _():
        pltpu.sync_copy(gA.at[pl.ds(0, HC)], h1_hbm.at[pl.ds(ebase, HC)])
        pltpu.sync_copy(gB.at[pl.ds(0, HC)], h1_hbm.at[pl.ds(ebase + HC, HC)])

    @pl.when(tid == NT - 1)
    def _():
        pltpu.sync_copy(gA.at[pl.ds(0, HC)], h1_hbm.at[pl.ds(ebase, HC)])
        pltpu.sync_copy(gB.at[pl.ds(0, RVAL - HC)],
                        h1_hbm.at[pl.ds(ebase + HC, RVAL - HC)])
    plsc.subcore_barrier()

    wbase = tid * NWE

    def start_idx(b, w):
        pltpu.async_copy(ed_hbm.at[cid, wbase + w], bufs[b][0], bufs[b][2])

    def wait_idx(b, w):
        pltpu.make_async_copy(ed_hbm.at[cid, wbase + w], bufs[b][0], bufs[b][2]).wait()

    def copy_rows(b):
        e_v, rowS = bufs[b][0], bufs[b][5]

        def body(i, c):
            cs = pl.ds(i * 16, 16)
            rowS[cs] = e_v[1, cs]
            return c
        lax.fori_loop(0, WE // 16, body, 0, unroll=5)

    def start_scatter(b):
        pltpu.async_copy(bufs[b][1], acc_sp.at[bufs[b][5]], bufs[b][4], add=True)

    def wait_scatter(b):
        pltpu.make_async_copy(bufs[b][1], acc_sp.at[bufs[b][5]], bufs[b][4]).wait()

    def mul(b):
        e_v, g_v = bufs[b][0], bufs[b][1]

        def body(g, c):
            w16 = plsc.bitcast(e_v[2, pl.ds(g * 16, 16)], jnp.float32)
            for j in range(16):
                wk = jnp.full((16,), w16[j], jnp.float32)
                k = g * 16 + j
                for ci in range(4):
                    cs = pl.ds(ci * 16, 16)
                    g_v[k, cs] = g_v[k, cs] * wk
            return c
        lax.fori_loop(0, WE // 16, body, 0, unroll=2)

    for L in range(3):
        h_src = (h1_hbm, h2_hbm, h1_hbm)[L]

        def start_gather(b, _h=h_src):
            pltpu.async_copy(_h.at[bufs[b][0].at[0]], bufs[b][1], bufs[b][3])

        def wait_gather(b, _h=h_src):
            pltpu.make_async_copy(_h.at[bufs[b][0].at[0]], bufs[b][1], bufs[b][3]).wait()

        def zrow(k, c):
            for ci in range(4):
                gA[k, pl.ds(ci * 16, 16)] = jnp.zeros((16,), jnp.float32)
            return c
        lax.fori_loop(0, WE, zrow, 0, unroll=4)
        pltpu.sync_copy(gA, acc_sp.at[pl.ds(r0, WE)])
        pltpu.sync_copy(gA.at[pl.ds(0, RPT - WE)],
                        acc_sp.at[pl.ds(r0 + WE, RPT - WE)])
        plsc.subcore_barrier()

        start_idx(0, 0)
        start_idx(1, 1)
        wait_idx(0, 0)
        start_gather(0)
        NH = NWE // 2

        def slots(hh, c, wait_gather=wait_gather, start_gather=start_gather):
            i0 = 2 * hh
            @pl.when(hh > 0)
            def _():
                wait_scatter(1)
            wait_idx(1, i0 + 1)
            start_gather(1)
            wait_gather(0)
            mul(0)
            copy_rows(0)
            start_scatter(0)

            @pl.when(hh < NH - 1)
            def _():
                start_idx(0, i0 + 2)
            wait_scatter(0)

            @pl.when(hh < NH - 1)
            def _():
                wait_idx(0, i0 + 2)
                start_gather(0)
            wait_gather(1)
            mul(1)
            copy_rows(1)
            start_scatter(1)

            @pl.when(hh < NH - 1)
            def _():
                start_idx(1, i0 + 3)
            return c
        lax.fori_loop(0, NH, slots, 0)
        wait_scatter(1)
        plsc.subcore_barrier()

        pltpu.sync_copy(deg_sp.at[pl.ds(r0, RPT)], inv_v)
        carry = ((z,) * 4, (z,) * 4)
        for sub in range(2):
            buf = bufs[sub][1]
            pltpu.sync_copy(acc_sp.at[pl.ds(r0 + sub * HC, HC)], buf.at[pl.ds(0, HC)])

            def srow(k, cr, _sub=sub, _buf=buf):
                s1s, s2s = cr
                f = jnp.where(r0 + _sub * HC + k < N, 1.0, 0.0)
                s = plsc.load_gather(
                    inv_v, [jnp.full((16,), _sub * HC + k, jnp.int32)]) * f
                n1, n2 = [], []
                for ci in range(4):
                    cs = pl.ds(ci * 16, 16)
                    t = _buf[k, cs] * s
                    _buf[k, cs] = t
                    n1.append(s1s[ci] + t)
                    n2.append(s2s[ci] + t * t)
                return tuple(n1), tuple(n2)
            carry = lax.fori_loop(0, HC, srow, carry, unroll=4)
        s1s, s2s = carry

        _stats_reduce(stats_sp, part_v, stat_v, s1s, s2s, tid)
        pltpu.sync_copy(gnp_hbm.at[pl.ds(16 + L * 16 + cid * 8, 8)], gn_v)

        alphas, betas, caps = [], [], []
        for c in range(4):
            cs = pl.ds(c * 16, 16)
            s1, s2 = stat_v[0, cs], stat_v[1, cs]
            w1, b1, ms1 = gn_v[0, cs], gn_v[1, cs], gn_v[2, cs]
            w2, b2, ms2 = gn_v[3, cs], gn_v[4, cs], gn_v[5, cs]
            cap = gn_v[6, cs]
            m = s1 * (1.0 / N)
            var1 = s2 * (1.0 / N) - m * m * ms1 * (2.0 - ms1)
            a1 = w1 * _rsqrt16(var1 + EPS)
            b1f = b1 - a1 * m * ms1
            m2 = a1 * m + b1f
            varh = s2 * (1.0 / N) - m * m
            mo = m2 * (1.0 - ms2)
            var2 = a1 * a1 * varh + mo * mo
            a2 = w2 * _rsqrt16(var2 + EPS)
            b2f = b2 - a2 * m2 * ms2
            alphas.append(a2 * a1)
            betas.append(a2 * b1f + b2f)
            caps.append(cap)

        for sub in range(2):
            buf = bufs[sub][1]

            def app(k, c, _buf=buf):
                for ci in range(4):
                    cs = pl.ds(ci * 16, 16)
                    t = _buf[k, cs] * alphas[ci] + betas[ci]
                    _buf[k, cs] = jnp.maximum(t, caps[ci])
                return c
            lax.fori_loop(0, HC, app, 0, unroll=4)

        if L < 2:
            h_dst = (h2_hbm, h1_hbm)[L]
            obase = cid * N + r0

            @pl.when(tid < NT - 1)
            def _(_h=h_dst, _b=obase):
                pltpu.sync_copy(gA.at[pl.ds(0, HC)], _h.at[pl.ds(_b, HC)])
                pltpu.sync_copy(gB.at[pl.ds(0, HC)], _h.at[pl.ds(_b + HC, HC)])

            @pl.when(tid == NT - 1)
            def _(_h=h_dst, _b=obase):
                pltpu.sync_copy(gA.at[pl.ds(0, HC)], _h.at[pl.ds(_b, HC)])
                pltpu.sync_copy(gB.at[pl.ds(0, RVAL - HC)],
                                _h.at[pl.ds(_b + HC, RVAL - HC)])
            plsc.subcore_barrier()
        else:
            @pl.when(tid < NT - 1)
            def _():
                pltpu.sync_copy(gA.at[pl.ds(0, HC)],
                                out_hbm.at[pl.ds(r0, HC), pl.ds(cid * HH, HH)])
                pltpu.sync_copy(gB.at[pl.ds(0, HC)],
                                out_hbm.at[pl.ds(r0 + HC, HC), pl.ds(cid * HH, HH)])

            @pl.when(tid == NT - 1)
            def _():
                pltpu.sync_copy(gA.at[pl.ds(0, HC)],
                                out_hbm.at[pl.ds(r0, HC), pl.ds(cid * HH, HH)])
                pltpu.sync_copy(gB.at[pl.ds(0, RVAL - HC)],
                                out_hbm.at[pl.ds(r0 + HC, RVAL - HC), pl.ds(cid * HH, HH)])


def _fused(x_pad, tbl2, row, ew, ed, gnp):
    return pl.kernel(
        _fused_body,
        out_type=(
            jax.ShapeDtypeStruct((N, H), jnp.float32),
            jax.ShapeDtypeStruct((2 * N, HH), jnp.float32),
            jax.ShapeDtypeStruct((2 * N, HH), jnp.float32),
        ),
        mesh=_mesh(),
        compiler_params=pltpu.CompilerParams(
            needs_layout_passes=False, use_tc_tiling_on_sc=False),
        scratch_types=[
            pltpu.VMEM((3, WE), jnp.int32),
            pltpu.VMEM((3, WE), jnp.int32),
            pltpu.VMEM((WE,), jnp.int32),
            pltpu.VMEM((WE,), jnp.int32),
            pltpu.VMEM((WE, HH), jnp.float32),
            pltpu.VMEM((WE, HH), jnp.float32),
            pltpu.VMEM((WD,), jnp.int32),
            pltpu.VMEM((WD,), jnp.float32),
            pltpu.VMEM((RPT,), jnp.float32),
            pltpu.VMEM((RPT,), jnp.int32),
            pltpu.VMEM((130, HH), jnp.float32),
            pltpu.VMEM((2 + 2 * NT, HH), jnp.float32),
            pltpu.VMEM((2, HH), jnp.float32),
            pltpu.VMEM((8, HH), jnp.float32),
            pltpu.VMEM_SHARED((NPAD, HH), jnp.float32),
            pltpu.VMEM_SHARED((NPAD,), jnp.float32),
            pltpu.VMEM_SHARED((2 * NT + 2, HH), jnp.float32),
            pltpu.SemaphoreType.DMA,
            pltpu.SemaphoreType.DMA,
            pltpu.SemaphoreType.DMA,
            pltpu.SemaphoreType.DMA,
            pltpu.SemaphoreType.DMA,
            pltpu.SemaphoreType.DMA,
        ],
    )(x_pad, tbl2, row, ew, ed, gnp)


def kernel(x, edge_index, edge_weight, emb_table, emb_gn, conv_gns, layer_gns):
    row = edge_index[0]
    col = edge_index[1]
    x_pad = jnp.concatenate([x, jnp.zeros((NPAD - N,), jnp.int32)])
    tbl2 = jnp.concatenate([emb_table[:, :HH], emb_table[:, HH:]], axis=0)
    zrow = jnp.zeros((HH,), jnp.float32)
    rows = [emb_gn["w"][:HH], emb_gn["b"][:HH], emb_gn["ms"][:HH]] + [zrow] * 5 \
        + [emb_gn["w"][HH:], emb_gn["b"][HH:], emb_gn["ms"][HH:]] + [zrow] * 5
    for layer in range(3):
        p1, p2 = conv_gns[layer], layer_gns[layer]
        cap = jnp.full((HH,), 0.0 if layer < 2 else -3.4e38, jnp.float32)
        for half in (slice(0, HH), slice(HH, 2 * HH)):
            for p in (p1, p2):
                rows += [p["w"][half], p["b"][half], p["ms"][half]]
            rows += [cap, zrow]
    gnp = jnp.stack(rows)

    ewbits = lax.bitcast_convert_type(edge_weight, jnp.int32)
    e0 = jnp.stack([col, row, ewbits])
    e1 = jnp.stack([col + jnp.int32(N), row, ewbits])
    ed = jnp.stack([e0, e1]).reshape(2, 3, E // WE, WE).transpose(0, 2, 1, 3)

    out, _, _ = _fused(x_pad, tbl2, row, edge_weight, ed, gnp)
    return out


# speedup vs baseline: 13.5329x; 1.0447x over previous
import jax
import jax.numpy as jnp
from jax import lax
from jax.experimental import pallas as pl
from jax.experimental.pallas import tpu as pltpu
from jax.experimental.pallas import tpu_sc as plsc

N = 10000
NPAD = 10240
E = 320000
H = 128
HH = 64
NT = 16
RPT = NPAD // NT
RVAL = N - 15 * RPT
EPT = E // NT
WD = 2000
WE = 400
NWD = EPT // WD
NWE = EPT // WE
EPS = 1e-6


def _mesh():
    return plsc.VectorSubcoreMesh(
        core_axis_name="c", subcore_axis_name="s", num_cores=2, num_subcores=16
    )


def _rsqrt16(v):
    i = lax.bitcast_convert_type(v, jnp.int32)
    i = jnp.int32(0x5F3759DF) - lax.shift_right_logical(i, 1)
    y = lax.bitcast_convert_type(i, jnp.float32)
    for _ in range(4):
        y = y * (1.5 - 0.5 * v * y * y)
    return y


def _stats_reduce(stats_sp, part_v, stat_v, s1s, s2s, tid):
    for c in range(4):
        part_v[0, pl.ds(c * 16, 16)] = s1s[c]
        part_v[1, pl.ds(c * 16, 16)] = s2s[c]
    pltpu.sync_copy(part_v.at[pl.ds(0, 1)], stats_sp.at[pl.ds(tid, 1)])
    pltpu.sync_copy(part_v.at[pl.ds(1, 1)], stats_sp.at[pl.ds(NT + tid, 1)])
    plsc.subcore_barrier()

    @pl.when(tid == 0)
    def _():
        pltpu.sync_copy(stats_sp.at[pl.ds(0, 2 * NT)], part_v.at[pl.ds(2, 2 * NT)])
        for c in range(4):
            def red(i, acc):
                a1, a2 = acc
                return (a1 + part_v[2 + i, pl.ds(c * 16, 16)],
                        a2 + part_v[2 + NT + i, pl.ds(c * 16, 16)])
            z = jnp.zeros((16,), jnp.float32)
            t1, t2 = lax.fori_loop(0, NT, red, (z, z))
            part_v[0, pl.ds(c * 16, 16)] = t1
            part_v[1, pl.ds(c * 16, 16)] = t2
        pltpu.sync_copy(part_v.at[pl.ds(0, 2)], stats_sp.at[pl.ds(2 * NT, 2)])
    plsc.subcore_barrier()
    pltpu.sync_copy(stats_sp.at[pl.ds(2 * NT, 2)], stat_v)


def _fused_body(x_hbm, tbl_hbm, row_hbm, ew_hbm, col2_hbm, rw_hbm, gnp_hbm,
                out_hbm, h1_hbm, h2_hbm,
                cA, cB, rwA, rwB, rowSA, rowSB, gA, gB, rowD, ewD,
                inv_v, xs_v, tbl_v, part_v, stat_v, gn_v,
                acc_sp, deg_sp, stats_sp,
                siA, siB, sgA, sgB, ssA, ssB):
    cid = lax.axis_index("c")
    tid = lax.axis_index("s")
    HC = RPT // 2
    r0 = tid * RPT

    bufs = ((cA, gA, siA, sgA, ssA, rowSA, rwA),
            (cB, gB, siB, sgB, ssB, rowSB, rwB))

    def zero16(i, c):
        inv_v[pl.ds(i * 16, 16)] = jnp.zeros((16,), jnp.float32)
        return c
    lax.fori_loop(0, RPT // 16, zero16, 0, unroll=4)
    pltpu.sync_copy(inv_v, deg_sp.at[pl.ds(r0, RPT)])
    plsc.subcore_barrier()

    def dwin(wi, c):
        dbase = tid * EPT + wi * WD
        pltpu.sync_copy(row_hbm.at[pl.ds(dbase, WD)], rowD)
        pltpu.sync_copy(ew_hbm.at[pl.ds(dbase, WD)], ewD)
        pltpu.sync_copy(ewD, deg_sp.at[rowD], add=True)
        return c
    lax.fori_loop(0, NWD, dwin, 0)
    plsc.subcore_barrier()

    pltpu.sync_copy(deg_sp.at[pl.ds(r0, RPT)], inv_v)

    def dinv(i, c):
        d = inv_v[pl.ds(i * 16, 16)]
        d = jnp.where(d < 0.5, d + 1.0, d)
        inv_v[pl.ds(i * 16, 16)] = 1.0 / d
        return c
    lax.fori_loop(0, RPT // 16, dinv, 0, unroll=4)
    pltpu.sync_copy(inv_v, deg_sp.at[pl.ds(r0, RPT)])

    pltpu.sync_copy(tbl_hbm, tbl_v)
    pltpu.sync_copy(x_hbm.at[pl.ds(r0, RPT)], xs_v)
    roff = cid * 65

    z = jnp.zeros((16,), jnp.float32)
    carry = ((z,) * 4, (z,) * 4)
    for sub in range(2):
        buf = bufs[sub][1]

        def enode(k, cr, _sub=sub, _buf=buf):
            s1s, s2s = cr
            f = jnp.where(r0 + _sub * HC + k < N, 1.0, 0.0)
            rowi = plsc.load_gather(
                xs_v, [jnp.full((16,), _sub * HC + k, jnp.int32)]) + roff
            n1, n2 = [], []
            for c in range(4):
                cols = lax.iota(jnp.int32, 16) + c * 16
                r = plsc.load_gather(tbl_v, [rowi, cols])
                _buf[k, pl.ds(c * 16, 16)] = r
                rf = r * f
                n1.append(s1s[c] + rf)
                n2.append(s2s[c] + rf * r)
            return tuple(n1), tuple(n2)
        carry = lax.fori_loop(0, HC, enode, carry, unroll=2)
    s1s, s2s = carry

    _stats_reduce(stats_sp, part_v, stat_v, s1s, s2s, tid)
    pltpu.sync_copy(gnp_hbm.at[pl.ds(cid * 8, 8)], gn_v)

    alphas, betas = [], []
    for c in range(4):
        cs = pl.ds(c * 16, 16)
        s1, s2 = stat_v[0, cs], stat_v[1, cs]
        w, b, ms = gn_v[0, cs], gn_v[1, cs], gn_v[2, cs]
        m = s1 * (1.0 / N)
        var = s2 * (1.0 / N) - m * m * ms * (2.0 - ms)
        a = w * _rsqrt16(var + EPS)
        alphas.append(a)
        betas.append(b - a * m * ms)

    for sub in range(2):
        buf = bufs[sub][1]

        def eapp(k, c, _buf=buf):
            for ci in range(4):
                cs = pl.ds(ci * 16, 16)
                _buf[k, cs] = _buf[k, cs] * alphas[ci] + betas[ci]
            return c
        lax.fori_loop(0, HC, eapp, 0, unroll=2)

    ebase = cid * N + r0

    @pl.when(tid < NT - 1)
    def _():
        pltpu.sync_copy(gA.at[pl.ds(0, HC)], h1_hbm.at[pl.ds(ebase, HC)])
        pltpu.sync_copy(gB.at[pl.ds(0, HC)], h1_hbm.at[pl.ds(ebase + HC, HC)])

    @pl.when(tid == NT - 1)
    def _():
        pltpu.sync_copy(gA.at[pl.ds(0, HC)], h1_hbm.at[pl.ds(ebase, HC)])
        pltpu.sync_copy(gB.at[pl.ds(0, RVAL - HC)],
                        h1_hbm.at[pl.ds(ebase + HC, RVAL - HC)])
    plsc.subcore_barrier()

    ebase2 = tid * EPT

    def start_idx(b, w):
        col_v, rw_v = bufs[b][0], bufs[b][6]
        si = bufs[b][2]
        pltpu.async_copy(col2_hbm.at[cid, pl.ds(ebase2 + w * WE, WE)], col_v, si)
        pltpu.async_copy(rw_hbm.at[pl.ds(0, 2), pl.ds(ebase2 + w * WE, WE)], rw_v, si)

    def wait_idx(b, w):
        col_v, rw_v = bufs[b][0], bufs[b][6]
        si = bufs[b][2]
        pltpu.make_async_copy(col2_hbm.at[cid, pl.ds(ebase2 + w * WE, WE)], col_v, si).wait()
        pltpu.make_async_copy(rw_hbm.at[pl.ds(0, 2), pl.ds(ebase2 + w * WE, WE)], rw_v, si).wait()

    def copy_rows(b):
        rw_v, rowS = bufs[b][6], bufs[b][5]

        def body(i, c):
            cs = pl.ds(i * 16, 16)
            rowS[cs] = rw_v[0, cs]
            return c
        lax.fori_loop(0, WE // 16, body, 0, unroll=5)

    def start_scatter(b):
        pltpu.async_copy(bufs[b][1], acc_sp.at[bufs[b][5]], bufs[b][4], add=True)

    def wait_scatter(b):
        pltpu.make_async_copy(bufs[b][1], acc_sp.at[bufs[b][5]], bufs[b][4]).wait()

    def mul(b):
        rw_v, g_v = bufs[b][6], bufs[b][1]

        def body(g, c):
            w16 = plsc.bitcast(rw_v[1, pl.ds(g * 16, 16)], jnp.float32)
            for j in range(16):
                wk = jnp.full((16,), w16[j], jnp.float32)
                k = g * 16 + j
                for ci in range(4):
                    cs = pl.ds(ci * 16, 16)
                    g_v[k, cs] = g_v[k, cs] * wk
            return c
        lax.fori_loop(0, WE // 16, body, 0, unroll=2)

    for L in range(3):
        h_src = (h1_hbm, h2_hbm, h1_hbm)[L]

        def start_gather(b, _h=h_src):
            pltpu.async_copy(_h.at[bufs[b][0]], bufs[b][1], bufs[b][3])

        def wait_gather(b, _h=h_src):
            pltpu.make_async_copy(_h.at[bufs[b][0]], bufs[b][1], bufs[b][3]).wait()

        def zrow(k, c):
            for ci in range(4):
                gA[k, pl.ds(ci * 16, 16)] = jnp.zeros((16,), jnp.float32)
            return c
        lax.fori_loop(0, WE, zrow, 0, unroll=4)
        pltpu.sync_copy(gA, acc_sp.at[pl.ds(r0, WE)])
        pltpu.sync_copy(gA.at[pl.ds(0, RPT - WE)],
                        acc_sp.at[pl.ds(r0 + WE, RPT - WE)])
        plsc.subcore_barrier()

        start_idx(0, 0)
        start_idx(1, 1)
        wait_idx(0, 0)
        start_gather(0)
        NH = NWE // 2

        def slots(hh, c, wait_gather=wait_gather, start_gather=start_gather):
            i0 = 2 * hh
            @pl.when(hh > 0)
            def _():
                wait_scatter(1)
            wait_idx(1, i0 + 1)
            start_gather(1)
            wait_gather(0)
            mul(0)
            copy_rows(0)
            start_scatter(0)

            @pl.when(hh < NH - 1)
            def _():
                start_idx(0, i0 + 2)
            wait_scatter(0)

            @pl.when(hh < NH - 1)
            def _():
                wait_idx(0, i0 + 2)
                start_gather(0)
            wait_gather(1)
            mul(1)
            copy_rows(1)
            start_scatter(1)

            @pl.when(hh < NH - 1)
            def _():
                start_idx(1, i0 + 3)
            return c
        lax.fori_loop(0, NH, slots, 0)
        wait_scatter(1)
        plsc.subcore_barrier()

        pltpu.sync_copy(deg_sp.at[pl.ds(r0, RPT)], inv_v)
        carry = ((z,) * 4, (z,) * 4)
        for sub in range(2):
            buf = bufs[sub][1]
            pltpu.sync_copy(acc_sp.at[pl.ds(r0 + sub * HC, HC)], buf.at[pl.ds(0, HC)])

            def srow(k, cr, _sub=sub, _buf=buf):
                s1s, s2s = cr
                f = jnp.where(r0 + _sub * HC + k < N, 1.0, 0.0)
                s = plsc.load_gather(
                    inv_v, [jnp.full((16,), _sub * HC + k, jnp.int32)]) * f
                n1, n2 = [], []
                for ci in range(4):
                    cs = pl.ds(ci * 16, 16)
                    t = _buf[k, cs] * s
                    _buf[k, cs] = t
                    n1.append(s1s[ci] + t)
                    n2.append(s2s[ci] + t * t)
                return tuple(n1), tuple(n2)
            carry = lax.fori_loop(0, HC, srow, carry, unroll=4)
        s1s, s2s = carry

        _stats_reduce(stats_sp, part_v, stat_v, s1s, s2s, tid)
        pltpu.sync_copy(gnp_hbm.at[pl.ds(16 + L * 16 + cid * 8, 8)], gn_v)

        alphas, betas, caps = [], [], []
        for c in range(4):
            cs = pl.ds(c * 16, 16)
            s1, s2 = stat_v[0, cs], stat_v[1, cs]
            w1, b1, ms1 = gn_v[0, cs], gn_v[1, cs], gn_v[2, cs]
            w2, b2, ms2 = gn_v[3, cs], gn_v[4, cs], gn_v[5, cs]
            cap = gn_v[6, cs]
            m = s1 * (1.0 / N)
            var1 = s2 * (1.0 / N) - m * m * ms1 * (2.0 - ms1)
            a1 = w1 * _rsqrt16(var1 + EPS)
            b1f = b1 - a1 * m * ms1
            m2 = a1 * m + b1f
            varh = s2 * (1.0 / N) - m * m
            mo = m2 * (1.0 - ms2)
            var2 = a1 * a1 * varh + mo * mo
            a2 = w2 * _rsqrt16(var2 + EPS)
            b2f = b2 - a2 * m2 * ms2
            alphas.append(a2 * a1)
            betas.append(a2 * b1f + b2f)
            caps.append(cap)

        for sub in range(2):
            buf = bufs[sub][1]

            def app(k, c, _buf=buf):
                for ci in range(4):
                    cs = pl.ds(ci * 16, 16)
                    t = _buf[k, cs] * alphas[ci] + betas[ci]
                    _buf[k, cs] = jnp.maximum(t, caps[ci])
                return c
            lax.fori_loop(0, HC, app, 0, unroll=4)

        if L < 2:
            h_dst = (h2_hbm, h1_hbm)[L]
            obase = cid * N + r0

            @pl.when(tid < NT - 1)
            def _(_h=h_dst, _b=obase):
                pltpu.sync_copy(gA.at[pl.ds(0, HC)], _h.at[pl.ds(_b, HC)])
                pltpu.sync_copy(gB.at[pl.ds(0, HC)], _h.at[pl.ds(_b + HC, HC)])

            @pl.when(tid == NT - 1)
            def _(_h=h_dst, _b=obase):
                pltpu.sync_copy(gA.at[pl.ds(0, HC)], _h.at[pl.ds(_b, HC)])
                pltpu.sync_copy(gB.at[pl.ds(0, RVAL - HC)],
                                _h.at[pl.ds(_b + HC, RVAL - HC)])
            plsc.subcore_barrier()
        else:
            @pl.when(tid < NT - 1)
            def _():
                pltpu.sync_copy(gA.at[pl.ds(0, HC)],
                                out_hbm.at[pl.ds(r0, HC), pl.ds(cid * HH, HH)])
                pltpu.sync_copy(gB.at[pl.ds(0, HC)],
                                out_hbm.at[pl.ds(r0 + HC, HC), pl.ds(cid * HH, HH)])

            @pl.when(tid == NT - 1)
            def _():
                pltpu.sync_copy(gA.at[pl.ds(0, HC)],
                                out_hbm.at[pl.ds(r0, HC), pl.ds(cid * HH, HH)])
                pltpu.sync_copy(gB.at[pl.ds(0, RVAL - HC)],
                                out_hbm.at[pl.ds(r0 + HC, RVAL - HC), pl.ds(cid * HH, HH)])


def _fused(x_pad, tbl2, row, ew, col2, rw, gnp):
    return pl.kernel(
        _fused_body,
        out_type=(
            jax.ShapeDtypeStruct((N, H), jnp.float32),
            jax.ShapeDtypeStruct((2 * N, HH), jnp.float32),
            jax.ShapeDtypeStruct((2 * N, HH), jnp.float32),
        ),
        mesh=_mesh(),
        compiler_params=pltpu.CompilerParams(
            needs_layout_passes=False, use_tc_tiling_on_sc=False),
        scratch_types=[
            pltpu.VMEM((WE,), jnp.int32),
            pltpu.VMEM((WE,), jnp.int32),
            pltpu.VMEM((2, WE), jnp.int32),
            pltpu.VMEM((2, WE), jnp.int32),
            pltpu.VMEM((WE,), jnp.int32),
            pltpu.VMEM((WE,), jnp.int32),
            pltpu.VMEM((WE, HH), jnp.float32),
            pltpu.VMEM((WE, HH), jnp.float32),
            pltpu.VMEM((WD,), jnp.int32),
            pltpu.VMEM((WD,), jnp.float32),
            pltpu.VMEM((RPT,), jnp.float32),
            pltpu.VMEM((RPT,), jnp.int32),
            pltpu.VMEM((130, HH), jnp.float32),
            pltpu.VMEM((2 + 2 * NT, HH), jnp.float32),
            pltpu.VMEM((2, HH), jnp.float32),
            pltpu.VMEM((8, HH), jnp.float32),
            pltpu.VMEM_SHARED((NPAD, HH), jnp.float32),
            pltpu.VMEM_SHARED((NPAD,), jnp.float32),
            pltpu.VMEM_SHARED((2 * NT + 2, HH), jnp.float32),
            pltpu.SemaphoreType.DMA,
            pltpu.SemaphoreType.DMA,
            pltpu.SemaphoreType.DMA,
            pltpu.SemaphoreType.DMA,
            pltpu.SemaphoreType.DMA,
            pltpu.SemaphoreType.DMA,
        ],
    )(x_pad, tbl2, row, ew, col2, rw, gnp)


def kernel(x, edge_index, edge_weight, emb_table, emb_gn, conv_gns, layer_gns):
    row = edge_index[0]
    col = edge_index[1]
    x_pad = jnp.concatenate([x, jnp.zeros((NPAD - N,), jnp.int32)])
    tbl2 = jnp.concatenate([emb_table[:, :HH], emb_table[:, HH:]], axis=0)
    zrow = jnp.zeros((HH,), jnp.float32)
    rows = [emb_gn["w"][:HH], emb_gn["b"][:HH], emb_gn["ms"][:HH]] + [zrow] * 5 \
        + [emb_gn["w"][HH:], emb_gn["b"][HH:], emb_gn["ms"][HH:]] + [zrow] * 5
    for layer in range(3):
        p1, p2 = conv_gns[layer], layer_gns[layer]
        cap = jnp.full((HH,), 0.0 if layer < 2 else -3.4e38, jnp.float32)
        for half in (slice(0, HH), slice(HH, 2 * HH)):
            for p in (p1, p2):
                rows += [p["w"][half], p["b"][half], p["ms"][half]]
            rows += [cap, zrow]
    gnp = jnp.stack(rows)

    ewbits = lax.bitcast_convert_type(edge_weight, jnp.int32)
    col2 = jnp.stack([col, col + jnp.int32(N)])
    rw = jnp.stack([row, ewbits])

    out, _, _ = _fused(x_pad, tbl2, row, edge_weight, col2, rw, gnp)
    return out


# speedup vs baseline: 14.2787x; 1.0551x over previous
import jax
import jax.numpy as jnp
from jax import lax
from jax.experimental import pallas as pl
from jax.experimental.pallas import tpu as pltpu
from jax.experimental.pallas import tpu_sc as plsc

N = 10000
NPAD = 10240
E = 320000
H = 128
HH = 64
NT = 16
RPT = NPAD // NT
RVAL = N - 15 * RPT
EPT = E // NT
WD = 2000
WE = 400
NWD = EPT // WD
NWE = EPT // WE
EPS = 1e-6


def _mesh():
    return plsc.VectorSubcoreMesh(
        core_axis_name="c", subcore_axis_name="s", num_cores=2, num_subcores=16
    )


def _rsqrt16(v):
    i = lax.bitcast_convert_type(v, jnp.int32)
    i = jnp.int32(0x5F3759DF) - lax.shift_right_logical(i, 1)
    y = lax.bitcast_convert_type(i, jnp.float32)
    for _ in range(4):
        y = y * (1.5 - 0.5 * v * y * y)
    return y


def _stats_reduce(stats_sp, part_v, stat_v, s1s, s2s, tid):
    for c in range(4):
        part_v[0, pl.ds(c * 16, 16)] = s1s[c]
        part_v[1, pl.ds(c * 16, 16)] = s2s[c]
    pltpu.sync_copy(part_v.at[pl.ds(0, 1)], stats_sp.at[pl.ds(tid, 1)])
    pltpu.sync_copy(part_v.at[pl.ds(1, 1)], stats_sp.at[pl.ds(NT + tid, 1)])
    plsc.subcore_barrier()

    @pl.when(tid == 0)
    def _():
        pltpu.sync_copy(stats_sp.at[pl.ds(0, 2 * NT)], part_v.at[pl.ds(2, 2 * NT)])
        for c in range(4):
            def red(i, acc):
                a1, a2 = acc
                return (a1 + part_v[2 + i, pl.ds(c * 16, 16)],
                        a2 + part_v[2 + NT + i, pl.ds(c * 16, 16)])
            z = jnp.zeros((16,), jnp.float32)
            t1, t2 = lax.fori_loop(0, NT, red, (z, z))
            part_v[0, pl.ds(c * 16, 16)] = t1
            part_v[1, pl.ds(c * 16, 16)] = t2
        pltpu.sync_copy(part_v.at[pl.ds(0, 2)], stats_sp.at[pl.ds(2 * NT, 2)])
    plsc.subcore_barrier()
    pltpu.sync_copy(stats_sp.at[pl.ds(2 * NT, 2)], stat_v)


def _fused_body(x_hbm, tbl_hbm, ei_hbm, ew_hbm, gnp_hbm,
                out_hbm, h1_hbm, h2_hbm,
                cA, cB, rwA, rwB, ewA, ewB, rowSA, rowSB, gA, gB, rowD, ewD,
                inv_v, xs_v, tbl_v, part_v, stat_v, gn_v,
                acc_sp, deg_sp, stats_sp,
                siA, siB, sgA, sgB, ssA, ssB):
    cid = lax.axis_index("c")
    tid = lax.axis_index("s")
    HC = RPT // 2
    r0 = tid * RPT

    bufs = ((cA, gA, siA, sgA, ssA, rowSA, rwA, ewA),
            (cB, gB, siB, sgB, ssB, rowSB, rwB, ewB))

    def zero16(i, c):
        inv_v[pl.ds(i * 16, 16)] = jnp.zeros((16,), jnp.float32)
        return c
    lax.fori_loop(0, RPT // 16, zero16, 0, unroll=4)
    pltpu.sync_copy(inv_v, deg_sp.at[pl.ds(r0, RPT)])
    plsc.subcore_barrier()

    def dwin(wi, c):
        dbase = tid * EPT + wi * WD
        pltpu.sync_copy(ei_hbm.at[0, pl.ds(dbase, WD)], rowD)
        pltpu.sync_copy(ew_hbm.at[pl.ds(dbase, WD)], ewD)
        pltpu.sync_copy(ewD, deg_sp.at[rowD], add=True)
        return c
    lax.fori_loop(0, NWD, dwin, 0)
    plsc.subcore_barrier()

    pltpu.sync_copy(deg_sp.at[pl.ds(r0, RPT)], inv_v)

    def dinv(i, c):
        d = inv_v[pl.ds(i * 16, 16)]
        d = jnp.where(d < 0.5, d + 1.0, d)
        inv_v[pl.ds(i * 16, 16)] = 1.0 / d
        return c
    lax.fori_loop(0, RPT // 16, dinv, 0, unroll=4)
    pltpu.sync_copy(inv_v, deg_sp.at[pl.ds(r0, RPT)])

    pltpu.sync_copy(tbl_hbm, tbl_v)
    pltpu.sync_copy(x_hbm.at[pl.ds(r0, RPT)], xs_v)
    roff = cid * 65

    z = jnp.zeros((16,), jnp.float32)
    carry = ((z,) * 4, (z,) * 4)
    for sub in range(2):
        buf = bufs[sub][1]

        def enode(k, cr, _sub=sub, _buf=buf):
            s1s, s2s = cr
            f = jnp.where(r0 + _sub * HC + k < N, 1.0, 0.0)
            rowi = plsc.load_gather(
                xs_v, [jnp.full((16,), _sub * HC + k, jnp.int32)]) + roff
            n1, n2 = [], []
            for c in range(4):
                cols = lax.iota(jnp.int32, 16) + c * 16
                r = plsc.load_gather(tbl_v, [rowi, cols])
                _buf[k, pl.ds(c * 16, 16)] = r
                rf = r * f
                n1.append(s1s[c] + rf)
                n2.append(s2s[c] + rf * r)
            return tuple(n1), tuple(n2)
        carry = lax.fori_loop(0, HC, enode, carry, unroll=2)
    s1s, s2s = carry

    _stats_reduce(stats_sp, part_v, stat_v, s1s, s2s, tid)
    pltpu.sync_copy(gnp_hbm.at[pl.ds(cid * 8, 8)], gn_v)

    alphas, betas = [], []
    for c in range(4):
        cs = pl.ds(c * 16, 16)
        s1, s2 = stat_v[0, cs], stat_v[1, cs]
        w, b, ms = gn_v[0, cs], gn_v[1, cs], gn_v[2, cs]
        m = s1 * (1.0 / N)
        var = s2 * (1.0 / N) - m * m * ms * (2.0 - ms)
        a = w * _rsqrt16(var + EPS)
        alphas.append(a)
        betas.append(b - a * m * ms)

    for sub in range(2):
        buf = bufs[sub][1]

        def eapp(k, c, _buf=buf):
            for ci in range(4):
                cs = pl.ds(ci * 16, 16)
                _buf[k, cs] = _buf[k, cs] * alphas[ci] + betas[ci]
            return c
        lax.fori_loop(0, HC, eapp, 0, unroll=2)

    ebase = cid * N + r0

    @pl.when(tid < NT - 1)
    def _():
        pltpu.sync_copy(gA.at[pl.ds(0, HC)], h1_hbm.at[pl.ds(ebase, HC)])
        pltpu.sync_copy(gB.at[pl.ds(0, HC)], h1_hbm.at[pl.ds(ebase + HC, HC)])

    @pl.when(tid == NT - 1)
    def _():
        pltpu.sync_copy(gA.at[pl.ds(0, HC)], h1_hbm.at[pl.ds(ebase, HC)])
        pltpu.sync_copy(gB.at[pl.ds(0, RVAL - HC)],
                        h1_hbm.at[pl.ds(ebase + HC, RVAL - HC)])
    plsc.subcore_barrier()

    ebase2 = tid * EPT
    coff = cid * N

    def start_idx(b, w):
        col_v, row_v, ew_v = bufs[b][0], bufs[b][6], bufs[b][7]
        si = bufs[b][2]
        pltpu.async_copy(ei_hbm.at[1, pl.ds(ebase2 + w * WE, WE)], col_v, si)
        pltpu.async_copy(ei_hbm.at[0, pl.ds(ebase2 + w * WE, WE)], row_v, si)
        pltpu.async_copy(ew_hbm.at[pl.ds(ebase2 + w * WE, WE)], ew_v, si)

    def wait_idx(b, w):
        col_v, row_v, ew_v = bufs[b][0], bufs[b][6], bufs[b][7]
        si = bufs[b][2]
        pltpu.make_async_copy(ei_hbm.at[1, pl.ds(ebase2 + w * WE, WE)], col_v, si).wait()
        pltpu.make_async_copy(ei_hbm.at[0, pl.ds(ebase2 + w * WE, WE)], row_v, si).wait()
        pltpu.make_async_copy(ew_hbm.at[pl.ds(ebase2 + w * WE, WE)], ew_v, si).wait()

        def adj(i, c):
            cs = pl.ds(i * 16, 16)
            col_v[cs] = col_v[cs] + coff
            return c
        lax.fori_loop(0, WE // 16, adj, 0, unroll=5)

    def copy_rows(b):
        row_v, rowS = bufs[b][6], bufs[b][5]

        def body(i, c):
            cs = pl.ds(i * 16, 16)
            rowS[cs] = row_v[cs]
            return c
        lax.fori_loop(0, WE // 16, body, 0, unroll=5)

    def start_scatter(b):
        pltpu.async_copy(bufs[b][1], acc_sp.at[bufs[b][5]], bufs[b][4], add=True)

    def wait_scatter(b):
        pltpu.make_async_copy(bufs[b][1], acc_sp.at[bufs[b][5]], bufs[b][4]).wait()

    def mul(b):
        ew_v, g_v = bufs[b][7], bufs[b][1]

        def body(g, c):
            w16 = ew_v[pl.ds(g * 16, 16)]
            for j in range(16):
                wk = jnp.full((16,), w16[j], jnp.float32)
                k = g * 16 + j
                for ci in range(4):
                    cs = pl.ds(ci * 16, 16)
                    g_v[k, cs] = g_v[k, cs] * wk
            return c
        lax.fori_loop(0, WE // 16, body, 0, unroll=2)

    for L in range(3):
        h_src = (h1_hbm, h2_hbm, h1_hbm)[L]

        def start_gather(b, _h=h_src):
            pltpu.async_copy(_h.at[bufs[b][0]], bufs[b][1], bufs[b][3])

        def wait_gather(b, _h=h_src):
            pltpu.make_async_copy(_h.at[bufs[b][0]], bufs[b][1], bufs[b][3]).wait()

        def zrow(k, c):
            for ci in range(4):
                gA[k, pl.ds(ci * 16, 16)] = jnp.zeros((16,), jnp.float32)
            return c
        lax.fori_loop(0, WE, zrow, 0, unroll=4)
        pltpu.sync_copy(gA, acc_sp.at[pl.ds(r0, WE)])
        pltpu.sync_copy(gA.at[pl.ds(0, RPT - WE)],
                        acc_sp.at[pl.ds(r0 + WE, RPT - WE)])
        plsc.subcore_barrier()

        start_idx(0, 0)
        start_idx(1, 1)
        wait_idx(0, 0)
        start_gather(0)
        NH = NWE // 2

        def slots(hh, c, wait_gather=wait_gather, start_gather=start_gather):
            i0 = 2 * hh
            @pl.when(hh > 0)
            def _():
                wait_scatter(1)
            wait_idx(1, i0 + 1)
            start_gather(1)
            wait_gather(0)
            mul(0)
            copy_rows(0)
            start_scatter(0)

            @pl.when(hh < NH - 1)
            def _():
                start_idx(0, i0 + 2)
            wait_scatter(0)

            @pl.when(hh < NH - 1)
            def _():
                wait_idx(0, i0 + 2)
                start_gather(0)
            wait_gather(1)
            mul(1)
            copy_rows(1)
            start_scatter(1)

            @pl.when(hh < NH - 1)
            def _():
                start_idx(1, i0 + 3)
            return c
        lax.fori_loop(0, NH, slots, 0)
        wait_scatter(1)
        plsc.subcore_barrier()

        pltpu.sync_copy(deg_sp.at[pl.ds(r0, RPT)], inv_v)
        carry = ((z,) * 4, (z,) * 4)
        for sub in range(2):
            buf = bufs[sub][1]
            pltpu.sync_copy(acc_sp.at[pl.ds(r0 + sub * HC, HC)], buf.at[pl.ds(0, HC)])

            def srow(k, cr, _sub=sub, _buf=buf):
                s1s, s2s = cr
                f = jnp.where(r0 + _sub * HC + k < N, 1.0, 0.0)
                s = plsc.load_gather(
                    inv_v, [jnp.full((16,), _sub * HC + k, jnp.int32)]) * f
                n1, n2 = [], []
                for ci in range(4):
                    cs = pl.ds(ci * 16, 16)
                    t = _buf[k, cs] * s
                    _buf[k, cs] = t
                    n1.append(s1s[ci] + t)
                    n2.append(s2s[ci] + t * t)
                return tuple(n1), tuple(n2)
            carry = lax.fori_loop(0, HC, srow, carry, unroll=4)
        s1s, s2s = carry

        _stats_reduce(stats_sp, part_v, stat_v, s1s, s2s, tid)
        pltpu.sync_copy(gnp_hbm.at[pl.ds(16 + L * 16 + cid * 8, 8)], gn_v)

        alphas, betas, caps = [], [], []
        for c in range(4):
            cs = pl.ds(c * 16, 16)
            s1, s2 = stat_v[0, cs], stat_v[1, cs]
            w1, b1, ms1 = gn_v[0, cs], gn_v[1, cs], gn_v[2, cs]
            w2, b2, ms2 = gn_v[3, cs], gn_v[4, cs], gn_v[5, cs]
            cap = gn_v[6, cs]
            m = s1 * (1.0 / N)
            var1 = s2 * (1.0 / N) - m * m * ms1 * (2.0 - ms1)
            a1 = w1 * _rsqrt16(var1 + EPS)
            b1f = b1 - a1 * m * ms1
            m2 = a1 * m + b1f
            varh = s2 * (1.0 / N) - m * m
            mo = m2 * (1.0 - ms2)
            var2 = a1 * a1 * varh + mo * mo
            a2 = w2 * _rsqrt16(var2 + EPS)
            b2f = b2 - a2 * m2 * ms2
            alphas.append(a2 * a1)
            betas.append(a2 * b1f + b2f)
            caps.append(cap)

        for sub in range(2):
            buf = bufs[sub][1]

            def app(k, c, _buf=buf):
                for ci in range(4):
                    cs = pl.ds(ci * 16, 16)
                    t = _buf[k, cs] * alphas[ci] + betas[ci]
                    _buf[k, cs] = jnp.maximum(t, caps[ci])
                return c
            lax.fori_loop(0, HC, app, 0, unroll=4)

        if L < 2:
            h_dst = (h2_hbm, h1_hbm)[L]
            obase = cid * N + r0

            @pl.when(tid < NT - 1)
            def _(_h=h_dst, _b=obase):
                pltpu.sync_copy(gA.at[pl.ds(0, HC)], _h.at[pl.ds(_b, HC)])
                pltpu.sync_copy(gB.at[pl.ds(0, HC)], _h.at[pl.ds(_b + HC, HC)])

            @pl.when(tid == NT - 1)
            def _(_h=h_dst, _b=obase):
                pltpu.sync_copy(gA.at[pl.ds(0, HC)], _h.at[pl.ds(_b, HC)])
                pltpu.sync_copy(gB.at[pl.ds(0, RVAL - HC)],
                                _h.at[pl.ds(_b + HC, RVAL - HC)])
            plsc.subcore_barrier()
        else:
            @pl.when(tid < NT - 1)
            def _():
                pltpu.sync_copy(gA.at[pl.ds(0, HC)],
                                out_hbm.at[pl.ds(r0, HC), pl.ds(cid * HH, HH)])
                pltpu.sync_copy(gB.at[pl.ds(0, HC)],
                                out_hbm.at[pl.ds(r0 + HC, HC), pl.ds(cid * HH, HH)])

            @pl.when(tid == NT - 1)
            def _():
                pltpu.sync_copy(gA.at[pl.ds(0, HC)],
                                out_hbm.at[pl.ds(r0, HC), pl.ds(cid * HH, HH)])
                pltpu.sync_copy(gB.at[pl.ds(0, RVAL - HC)],
                                out_hbm.at[pl.ds(r0 + HC, RVAL - HC), pl.ds(cid * HH, HH)])


def _fused(x_pad, tbl2, ei, ew, gnp):
    return pl.kernel(
        _fused_body,
        out_type=(
            jax.ShapeDtypeStruct((N, H), jnp.float32),
            jax.ShapeDtypeStruct((2 * N, HH), jnp.float32),
            jax.ShapeDtypeStruct((2 * N, HH), jnp.float32),
        ),
        mesh=_mesh(),
        compiler_params=pltpu.CompilerParams(
            needs_layout_passes=False, use_tc_tiling_on_sc=False),
        scratch_types=[
            pltpu.VMEM((WE,), jnp.int32),
            pltpu.VMEM((WE,), jnp.int32),
            pltpu.VMEM((WE,), jnp.int32),
            pltpu.VMEM((WE,), jnp.int32),
            pltpu.VMEM((WE,), jnp.float32),
            pltpu.VMEM((WE,), jnp.float32),
            pltpu.VMEM((WE,), jnp.int32),
            pltpu.VMEM((WE,), jnp.int32),
            pltpu.VMEM((WE, HH), jnp.float32),
            pltpu.VMEM((WE, HH), jnp.float32),
            pltpu.VMEM((WD,), jnp.int32),
            pltpu.VMEM((WD,), jnp.float32),
            pltpu.VMEM((RPT,), jnp.float32),
            pltpu.VMEM((RPT,), jnp.int32),
            pltpu.VMEM((130, HH), jnp.float32),
            pltpu.VMEM((2 + 2 * NT, HH), jnp.float32),
            pltpu.VMEM((2, HH), jnp.float32),
            pltpu.VMEM((8, HH), jnp.float32),
            pltpu.VMEM_SHARED((NPAD, HH), jnp.float32),
            pltpu.VMEM_SHARED((NPAD,), jnp.float32),
            pltpu.VMEM_SHARED((2 * NT + 2, HH), jnp.float32),
            pltpu.SemaphoreType.DMA,
            pltpu.SemaphoreType.DMA,
            pltpu.SemaphoreType.DMA,
            pltpu.SemaphoreType.DMA,
            pltpu.SemaphoreType.DMA,
            pltpu.SemaphoreType.DMA,
        ],
    )(x_pad, tbl2, ei, ew, gnp)


def kernel(x, edge_index, edge_weight, emb_table, emb_gn, conv_gns, layer_gns):
    x_pad = jnp.concatenate([x, jnp.zeros((NPAD - N,), jnp.int32)])
    tbl2 = jnp.concatenate([emb_table[:, :HH], emb_table[:, HH:]], axis=0)
    zrow = jnp.zeros((HH,), jnp.float32)
    rows = [emb_gn["w"][:HH], emb_gn["b"][:HH], emb_gn["ms"][:HH]] + [zrow] * 5 \
        + [emb_gn["w"][HH:], emb_gn["b"][HH:], emb_gn["ms"][HH:]] + [zrow] * 5
    for layer in range(3):
        p1, p2 = conv_gns[layer], layer_gns[layer]
        cap = jnp.full((HH,), 0.0 if layer < 2 else -3.4e38, jnp.float32)
        for half in (slice(0, HH), slice(HH, 2 * HH)):
            for p in (p1, p2):
                rows += [p["w"][half], p["b"][half], p["ms"][half]]
            rows += [cap, zrow]
    gnp = jnp.stack(rows)

    out, _, _ = _fused(x_pad, tbl2, edge_index, edge_weight, gnp)
    return out


# speedup vs baseline: 14.3741x; 1.0067x over previous
import jax
import jax.numpy as jnp
from jax import lax
from jax.experimental import pallas as pl
from jax.experimental.pallas import tpu as pltpu
from jax.experimental.pallas import tpu_sc as plsc

N = 10000
NPAD = 10240
E = 320000
H = 128
HH = 64
NT = 16
RPT = NPAD // NT
RVAL = N - 15 * RPT
EPT = E // NT
WD = 4000
WE = 400
NWD = EPT // WD
NWE = EPT // WE
EPS = 1e-6


def _mesh():
    return plsc.VectorSubcoreMesh(
        core_axis_name="c", subcore_axis_name="s", num_cores=2, num_subcores=16
    )


def _rsqrt16(v):
    i = lax.bitcast_convert_type(v, jnp.int32)
    i = jnp.int32(0x5F3759DF) - lax.shift_right_logical(i, 1)
    y = lax.bitcast_convert_type(i, jnp.float32)
    for _ in range(4):
        y = y * (1.5 - 0.5 * v * y * y)
    return y


def _stats_reduce(stats_sp, part_v, stat_v, s1s, s2s, tid):
    for c in range(4):
        part_v[0, pl.ds(c * 16, 16)] = s1s[c]
        part_v[1, pl.ds(c * 16, 16)] = s2s[c]
    pltpu.sync_copy(part_v.at[pl.ds(0, 1)], stats_sp.at[pl.ds(tid, 1)])
    pltpu.sync_copy(part_v.at[pl.ds(1, 1)], stats_sp.at[pl.ds(NT + tid, 1)])
    plsc.subcore_barrier()

    @pl.when(tid == 0)
    def _():
        pltpu.sync_copy(stats_sp.at[pl.ds(0, 2 * NT)], part_v.at[pl.ds(2, 2 * NT)])
        for c in range(4):
            def red(i, acc):
                a1, a2 = acc
                return (a1 + part_v[2 + i, pl.ds(c * 16, 16)],
                        a2 + part_v[2 + NT + i, pl.ds(c * 16, 16)])
            z = jnp.zeros((16,), jnp.float32)
            t1, t2 = lax.fori_loop(0, NT, red, (z, z))
            part_v[0, pl.ds(c * 16, 16)] = t1
            part_v[1, pl.ds(c * 16, 16)] = t2
        pltpu.sync_copy(part_v.at[pl.ds(0, 2)], stats_sp.at[pl.ds(2 * NT, 2)])
    plsc.subcore_barrier()
    pltpu.sync_copy(stats_sp.at[pl.ds(2 * NT, 2)], stat_v)


def _fused_body(x_hbm, tbl_hbm, ei_hbm, ew_hbm, gnp_hbm,
                out_hbm, h1_hbm, h2_hbm,
                cA, cB, rwA, rwB, ewA, ewB, rowSA, rowSB, gA, gB, rowD, ewD,
                inv_v, xs_v, tbl_v, part_v, stat_v, gn_v,
                acc_sp, deg_sp, stats_sp,
                siA, siB, sgA, sgB, ssA, ssB):
    cid = lax.axis_index("c")
    tid = lax.axis_index("s")
    HC = RPT // 2
    r0 = tid * RPT

    bufs = ((cA, gA, siA, sgA, ssA, rowSA, rwA, ewA),
            (cB, gB, siB, sgB, ssB, rowSB, rwB, ewB))

    def zero16(i, c):
        inv_v[pl.ds(i * 16, 16)] = jnp.zeros((16,), jnp.float32)
        return c
    lax.fori_loop(0, RPT // 16, zero16, 0, unroll=4)
    pltpu.sync_copy(inv_v, deg_sp.at[pl.ds(r0, RPT)])
    plsc.subcore_barrier()

    def dwin(wi, c):
        dbase = tid * EPT + wi * WD
        pltpu.sync_copy(ei_hbm.at[0, pl.ds(dbase, WD)], rowD)
        pltpu.sync_copy(ew_hbm.at[pl.ds(dbase, WD)], ewD)
        pltpu.sync_copy(ewD, deg_sp.at[rowD], add=True)
        return c
    lax.fori_loop(0, NWD, dwin, 0)
    plsc.subcore_barrier()

    pltpu.sync_copy(deg_sp.at[pl.ds(r0, RPT)], inv_v)

    def dinv(i, c):
        d = inv_v[pl.ds(i * 16, 16)]
        d = jnp.where(d < 0.5, d + 1.0, d)
        inv_v[pl.ds(i * 16, 16)] = 1.0 / d
        return c
    lax.fori_loop(0, RPT // 16, dinv, 0, unroll=4)
    pltpu.sync_copy(inv_v, deg_sp.at[pl.ds(r0, RPT)])

    pltpu.sync_copy(tbl_hbm, tbl_v)
    pltpu.sync_copy(x_hbm.at[pl.ds(r0, RPT)], xs_v)
    roff = cid * 65

    z = jnp.zeros((16,), jnp.float32)
    carry = ((z,) * 4, (z,) * 4)
    for sub in range(2):
        buf = bufs[sub][1]

        def enode(k, cr, _sub=sub, _buf=buf):
            s1s, s2s = cr
            f = jnp.where(r0 + _sub * HC + k < N, 1.0, 0.0)
            rowi = plsc.load_gather(
                xs_v, [jnp.full((16,), _sub * HC + k, jnp.int32)]) + roff
            n1, n2 = [], []
            for c in range(4):
                cols = lax.iota(jnp.int32, 16) + c * 16
                r = plsc.load_gather(tbl_v, [rowi, cols])
                _buf[k, pl.ds(c * 16, 16)] = r
                rf = r * f
                n1.append(s1s[c] + rf)
                n2.append(s2s[c] + rf * r)
            return tuple(n1), tuple(n2)
        carry = lax.fori_loop(0, HC, enode, carry, unroll=2)
    s1s, s2s = carry

    _stats_reduce(stats_sp, part_v, stat_v, s1s, s2s, tid)
    pltpu.sync_copy(gnp_hbm.at[pl.ds(cid * 8, 8)], gn_v)

    alphas, betas = [], []
    for c in range(4):
        cs = pl.ds(c * 16, 16)
        s1, s2 = stat_v[0, cs], stat_v[1, cs]
        w, b, ms = gn_v[0, cs], gn_v[1, cs], gn_v[2, cs]
        m = s1 * (1.0 / N)
        var = s2 * (1.0 / N) - m * m * ms * (2.0 - ms)
        a = w * _rsqrt16(var + EPS)
        alphas.append(a)
        betas.append(b - a * m * ms)

    for sub in range(2):
        buf = bufs[sub][1]

        def eapp(k, c, _buf=buf):
            for ci in range(4):
                cs = pl.ds(ci * 16, 16)
                _buf[k, cs] = _buf[k, cs] * alphas[ci] + betas[ci]
            return c
        lax.fori_loop(0, HC, eapp, 0, unroll=2)

    ebase = cid * N + r0

    @pl.when(tid < NT - 1)
    def _():
        pltpu.sync_copy(gA.at[pl.ds(0, HC)], h1_hbm.at[pl.ds(ebase, HC)])
        pltpu.sync_copy(gB.at[pl.ds(0, HC)], h1_hbm.at[pl.ds(ebase + HC, HC)])

    @pl.when(tid == NT - 1)
    def _():
        pltpu.sync_copy(gA.at[pl.ds(0, HC)], h1_hbm.at[pl.ds(ebase, HC)])
        pltpu.sync_copy(gB.at[pl.ds(0, RVAL - HC)],
                        h1_hbm.at[pl.ds(ebase + HC, RVAL - HC)])
    plsc.subcore_barrier()

    ebase2 = tid * EPT
    coff = cid * N

    def start_idx(b, w):
        col_v, row_v, ew_v = bufs[b][0], bufs[b][6], bufs[b][7]
        si = bufs[b][2]
        pltpu.async_copy(ei_hbm.at[1, pl.ds(ebase2 + w * WE, WE)], col_v, si)
        pltpu.async_copy(ei_hbm.at[0, pl.ds(ebase2 + w * WE, WE)], row_v, si)
        pltpu.async_copy(ew_hbm.at[pl.ds(ebase2 + w * WE, WE)], ew_v, si)

    def wait_idx(b, w):
        col_v, row_v, ew_v = bufs[b][0], bufs[b][6], bufs[b][7]
        si = bufs[b][2]
        pltpu.make_async_copy(ei_hbm.at[1, pl.ds(ebase2 + w * WE, WE)], col_v, si).wait()
        pltpu.make_async_copy(ei_hbm.at[0, pl.ds(ebase2 + w * WE, WE)], row_v, si).wait()
        pltpu.make_async_copy(ew_hbm.at[pl.ds(ebase2 + w * WE, WE)], ew_v, si).wait()

        def adj(i, c):
            cs = pl.ds(i * 16, 16)
            col_v[cs] = col_v[cs] + coff
            return c
        lax.fori_loop(0, WE // 16, adj, 0, unroll=5)

    def copy_rows(b):
        row_v, rowS = bufs[b][6], bufs[b][5]

        def body(i, c):
            cs = pl.ds(i * 16, 16)
            rowS[cs] = row_v[cs]
            return c
        lax.fori_loop(0, WE // 16, body, 0, unroll=5)

    def start_scatter(b):
        pltpu.async_copy(bufs[b][1], acc_sp.at[bufs[b][5]], bufs[b][4], add=True)

    def wait_scatter(b):
        pltpu.make_async_copy(bufs[b][1], acc_sp.at[bufs[b][5]], bufs[b][4]).wait()

    def mul(b):
        ew_v, g_v = bufs[b][7], bufs[b][1]

        def body(g, c):
            w16 = ew_v[pl.ds(g * 16, 16)]
            for j in range(16):
                wk = jnp.full((16,), w16[j], jnp.float32)
                k = g * 16 + j
                for ci in range(4):
                    cs = pl.ds(ci * 16, 16)
                    g_v[k, cs] = g_v[k, cs] * wk
            return c
        lax.fori_loop(0, WE // 16, body, 0, unroll=4)

    for L in range(3):
        h_src = (h1_hbm, h2_hbm, h1_hbm)[L]

        def start_gather(b, _h=h_src):
            pltpu.async_copy(_h.at[bufs[b][0]], bufs[b][1], bufs[b][3])

        def wait_gather(b, _h=h_src):
            pltpu.make_async_copy(_h.at[bufs[b][0]], bufs[b][1], bufs[b][3]).wait()

        def zrow(k, c):
            for ci in range(4):
                gA[k, pl.ds(ci * 16, 16)] = jnp.zeros((16,), jnp.float32)
            return c
        lax.fori_loop(0, WE, zrow, 0, unroll=4)
        pltpu.sync_copy(gA, acc_sp.at[pl.ds(r0, WE)])
        pltpu.sync_copy(gA.at[pl.ds(0, RPT - WE)],
                        acc_sp.at[pl.ds(r0 + WE, RPT - WE)])
        plsc.subcore_barrier()

        start_idx(0, 0)
        start_idx(1, 1)
        wait_idx(0, 0)
        start_gather(0)
        NH = NWE // 2

        def slots(hh, c, wait_gather=wait_gather, start_gather=start_gather):
            i0 = 2 * hh
            @pl.when(hh > 0)
            def _():
                wait_scatter(1)
            wait_idx(1, i0 + 1)
            start_gather(1)
            wait_gather(0)
            mul(0)
            copy_rows(0)
            start_scatter(0)

            @pl.when(hh < NH - 1)
            def _():
                start_idx(0, i0 + 2)
            wait_scatter(0)

            @pl.when(hh < NH - 1)
            def _():
                wait_idx(0, i0 + 2)
                start_gather(0)
            wait_gather(1)
            mul(1)
            copy_rows(1)
            start_scatter(1)

            @pl.when(hh < NH - 1)
            def _():
                start_idx(1, i0 + 3)
            return c
        lax.fori_loop(0, NH, slots, 0)
        wait_scatter(1)
        plsc.subcore_barrier()

        pltpu.sync_copy(deg_sp.at[pl.ds(r0, RPT)], inv_v)
        carry = ((z,) * 4, (z,) * 4)
        for sub in range(2):
            buf = bufs[sub][1]
            pltpu.sync_copy(acc_sp.at[pl.ds(r0 + sub * HC, HC)], buf.at[pl.ds(0, HC)])

            def srow(k, cr, _sub=sub, _buf=buf):
                s1s, s2s = cr
                f = jnp.where(r0 + _sub * HC + k < N, 1.0, 0.0)
                s = plsc.load_gather(
                    inv_v, [jnp.full((16,), _sub * HC + k, jnp.int32)]) * f
                n1, n2 = [], []
                for ci in range(4):
                    cs = pl.ds(ci * 16, 16)
                    t = _buf[k, cs] * s
                    _buf[k, cs] = t
                    n1.append(s1s[ci] + t)
                    n2.append(s2s[ci] + t * t)
                return tuple(n1), tuple(n2)
            carry = lax.fori_loop(0, HC, srow, carry, unroll=4)
        s1s, s2s = carry

        _stats_reduce(stats_sp, part_v, stat_v, s1s, s2s, tid)
        pltpu.sync_copy(gnp_hbm.at[pl.ds(16 + L * 16 + cid * 8, 8)], gn_v)

        alphas, betas, caps = [], [], []
        for c in range(4):
            cs = pl.ds(c * 16, 16)
            s1, s2 = stat_v[0, cs], stat_v[1, cs]
            w1, b1, ms1 = gn_v[0, cs], gn_v[1, cs], gn_v[2, cs]
            w2, b2, ms2 = gn_v[3, cs], gn_v[4, cs], gn_v[5, cs]
            cap = gn_v[6, cs]
            m = s1 * (1.0 / N)
            var1 = s2 * (1.0 / N) - m * m * ms1 * (2.0 - ms1)
            a1 = w1 * _rsqrt16(var1 + EPS)
            b1f = b1 - a1 * m * ms1
            m2 = a1 * m + b1f
            varh = s2 * (1.0 / N) - m * m
            mo = m2 * (1.0 - ms2)
            var2 = a1 * a1 * varh + mo * mo
            a2 = w2 * _rsqrt16(var2 + EPS)
            b2f = b2 - a2 * m2 * ms2
            alphas.append(a2 * a1)
            betas.append(a2 * b1f + b2f)
            caps.append(cap)

        for sub in range(2):
            buf = bufs[sub][1]

            def app(k, c, _buf=buf):
                for ci in range(4):
                    cs = pl.ds(ci * 16, 16)
                    t = _buf[k, cs] * alphas[ci] + betas[ci]
                    _buf[k, cs] = jnp.maximum(t, caps[ci])
                return c
            lax.fori_loop(0, HC, app, 0, unroll=4)

        if L < 2:
            h_dst = (h2_hbm, h1_hbm)[L]
            obase = cid * N + r0

            @pl.when(tid < NT - 1)
            def _(_h=h_dst, _b=obase):
                pltpu.sync_copy(gA.at[pl.ds(0, HC)], _h.at[pl.ds(_b, HC)])
                pltpu.sync_copy(gB.at[pl.ds(0, HC)], _h.at[pl.ds(_b + HC, HC)])

            @pl.when(tid == NT - 1)
            def _(_h=h_dst, _b=obase):
                pltpu.sync_copy(gA.at[pl.ds(0, HC)], _h.at[pl.ds(_b, HC)])
                pltpu.sync_copy(gB.at[pl.ds(0, RVAL - HC)],
                                _h.at[pl.ds(_b + HC, RVAL - HC)])
            plsc.subcore_barrier()
        else:
            @pl.when(tid < NT - 1)
            def _():
                pltpu.sync_copy(gA.at[pl.ds(0, HC)],
                                out_hbm.at[pl.ds(r0, HC), pl.ds(cid * HH, HH)])
                pltpu.sync_copy(gB.at[pl.ds(0, HC)],
                                out_hbm.at[pl.ds(r0 + HC, HC), pl.ds(cid * HH, HH)])

            @pl.when(tid == NT - 1)
            def _():
                pltpu.sync_copy(gA.at[pl.ds(0, HC)],
                                out_hbm.at[pl.ds(r0, HC), pl.ds(cid * HH, HH)])
                pltpu.sync_copy(gB.at[pl.ds(0, RVAL - HC)],
                                out_hbm.at[pl.ds(r0 + HC, RVAL - HC), pl.ds(cid * HH, HH)])


def _fused(x_pad, tbl2, ei, ew, gnp):
    return pl.kernel(
        _fused_body,
        out_type=(
            jax.ShapeDtypeStruct((N, H), jnp.float32),
            jax.ShapeDtypeStruct((2 * N, HH), jnp.float32),
            jax.ShapeDtypeStruct((2 * N, HH), jnp.float32),
        ),
        mesh=_mesh(),
        compiler_params=pltpu.CompilerParams(
            needs_layout_passes=False, use_tc_tiling_on_sc=False),
        scratch_types=[
            pltpu.VMEM((WE,), jnp.int32),
            pltpu.VMEM((WE,), jnp.int32),
            pltpu.VMEM((WE,), jnp.int32),
            pltpu.VMEM((WE,), jnp.int32),
            pltpu.VMEM((WE,), jnp.float32),
            pltpu.VMEM((WE,), jnp.float32),
            pltpu.VMEM((WE,), jnp.int32),
            pltpu.VMEM((WE,), jnp.int32),
            pltpu.VMEM((WE, HH), jnp.float32),
            pltpu.VMEM((WE, HH), jnp.float32),
            pltpu.VMEM((WD,), jnp.int32),
            pltpu.VMEM((WD,), jnp.float32),
            pltpu.VMEM((RPT,), jnp.float32),
            pltpu.VMEM((RPT,), jnp.int32),
            pltpu.VMEM((130, HH), jnp.float32),
            pltpu.VMEM((2 + 2 * NT, HH), jnp.float32),
            pltpu.VMEM((2, HH), jnp.float32),
            pltpu.VMEM((8, HH), jnp.float32),
            pltpu.VMEM_SHARED((NPAD, HH), jnp.float32),
            pltpu.VMEM_SHARED((NPAD,), jnp.float32),
            pltpu.VMEM_SHARED((2 * NT + 2, HH), jnp.float32),
            pltpu.SemaphoreType.DMA,
            pltpu.SemaphoreType.DMA,
            pltpu.SemaphoreType.DMA,
            pltpu.SemaphoreType.DMA,
            pltpu.SemaphoreType.DMA,
            pltpu.SemaphoreType.DMA,
        ],
    )(x_pad, tbl2, ei, ew, gnp)


def kernel(x, edge_index, edge_weight, emb_table, emb_gn, conv_gns, layer_gns):
    x_pad = jnp.concatenate([x, jnp.zeros((NPAD - N,), jnp.int32)])
    tbl2 = jnp.concatenate([emb_table[:, :HH], emb_table[:, HH:]], axis=0)
    zrow = jnp.zeros((HH,), jnp.float32)
    rows = [emb_gn["w"][:HH], emb_gn["b"][:HH], emb_gn["ms"][:HH]] + [zrow] * 5 \
        + [emb_gn["w"][HH:], emb_gn["b"][HH:], emb_gn["ms"][HH:]] + [zrow] * 5
    for layer in range(3):
        p1, p2 = conv_gns[layer], layer_gns[layer]
        cap = jnp.full((HH,), 0.0 if layer < 2 else -3.4e38, jnp.float32)
        for half in (slice(0, HH), slice(HH, 2 * HH)):
            for p in (p1, p2):
                rows += [p["w"][half], p["b"][half], p["ms"][half]]
            rows += [cap, zrow]
    gnp = jnp.stack(rows)

    out, _, _ = _fused(x_pad, tbl2, edge_index, edge_weight, gnp)
    return out
